# baseline (speedup 1.0000x reference)
import functools

import jax
import jax.numpy as jnp
from jax import lax
from jax.experimental import pallas as pl
from jax.experimental.pallas import tpu as pltpu
from jax.experimental.pallas import tpu_sc as plsc

N = 10000
M = 10000
E = 320000
H = 128

_NC = 2
_NS = 16
_NW = _NC * _NS
_CHUNK = 125
_ROWS_PER_W = E // _CHUNK // _NW
_IDX_BLOCK = 40


def _lrelu(v):
    return jnp.where(v >= 0, v, 0.01 * v)


def _graph_norm(x, gamma, beta, alpha, eps=1e-5):
    mean = jnp.mean(x, axis=0, keepdims=True)
    xc = x - alpha * mean
    var = jnp.mean(xc * xc, axis=0, keepdims=True)
    return gamma * xc / jnp.sqrt(var + eps) + beta


def _seg_sum_body(table, gidx, sidx, zeros, out, gidx_v, sidx_v, rows0, rows1,
                  acc, gsem0, gsem1):
    c = lax.axis_index("c")
    s = lax.axis_index("s")
    w = s * _NC + c

    rows_per_sub = 624
    pltpu.sync_copy(zeros.at[pl.ds(s * rows_per_sub, rows_per_sub)],
                    acc.at[pl.ds(s * rows_per_sub, rows_per_sub)])

    @pl.when(s == 0)
    def _():
        pltpu.sync_copy(zeros.at[pl.ds(_NS * rows_per_sub, M - _NS * rows_per_sub)],
                        acc.at[pl.ds(_NS * rows_per_sub, M - _NS * rows_per_sub)])
    plsc.subcore_barrier()

    rows = (rows0, rows1)
    sems = (gsem0, gsem1)
    descs = [None, None]

    def start(j, b):
        descs[b] = pltpu.async_copy(table.at[gidx_v.at[j]], rows[b], sems[b])

    for blk in range(_ROWS_PER_W // _IDX_BLOCK):
        base = w * _ROWS_PER_W + blk * _IDX_BLOCK
        pltpu.sync_copy(gidx.at[pl.ds(base, _IDX_BLOCK)], gidx_v)
        pltpu.sync_copy(sidx.at[pl.ds(base, _IDX_BLOCK)], sidx_v)
        start(0, 0)
        for j in range(_IDX_BLOCK):
            b = j & 1
            if j + 1 < _IDX_BLOCK:
                start(j + 1, (j + 1) & 1)
            descs[b].wait()
            pltpu.sync_copy(rows[b], acc.at[sidx_v.at[j]], add=True)

    plsc.subcore_barrier()
    pltpu.sync_copy(acc.at[pl.ds(s * rows_per_sub, rows_per_sub)],
                    out.at[c].at[pl.ds(s * rows_per_sub, rows_per_sub)])

    @pl.when(s == 0)
    def _():
        pltpu.sync_copy(acc.at[pl.ds(_NS * rows_per_sub, M - _NS * rows_per_sub)],
                        out.at[c].at[pl.ds(_NS * rows_per_sub, M - _NS * rows_per_sub)])


_EPW = E // _NW
_NV = _EPW // 16
_SLOTS = 8


def _attn_ex_body(sq, sk, srcf, dstf, bv, ex_out, denp_out,
                  sq_v, sk_v, src_v, dst_v, ex_v, b_v, dacc):
    c = lax.axis_index("c")
    s = lax.axis_index("s")
    w = s * _NC + c
    base = w * _EPW
    pltpu.sync_copy(sq, sq_v)
    pltpu.sync_copy(sk, sk_v)
    pltpu.sync_copy(srcf.at[pl.ds(base, _EPW)], src_v)
    pltpu.sync_copy(dstf.at[pl.ds(base, _EPW)], dst_v)
    pltpu.sync_copy(bv, b_v)
    b16 = b_v[...]
    iota = lax.iota(jnp.int32, 16)
    mlo = iota < _SLOTS
    mhi = jnp.logical_not(mlo)
    slot_off = (iota % _SLOTS) * N

    def zbody(v, _):
        dacc[pl.ds(v * 16, 16)] = jnp.zeros((16,), jnp.float32)
        return 0

    lax.fori_loop(0, _SLOTS * N // 16, zbody, 0)

    def body(v, _):
        s16 = src_v[pl.ds(v * 16, 16)]
        d16 = dst_v[pl.ds(v * 16, 16)]
        a = plsc.load_gather(sq_v, [s16])
        b = plsc.load_gather(sk_v, [d16])
        sc = a + b
        sc = jnp.where(sc >= 0, sc, 0.01 * sc)
        e = jnp.exp(sc - b16)
        ex_v[pl.ds(v * 16, 16)] = e
        didx = slot_off + s16
        plsc.addupdate_scatter(dacc, [didx], e, mask=mlo)
        plsc.addupdate_scatter(dacc, [didx], e, mask=mhi)
        return 0

    lax.fori_loop(0, _NV, body, 0)

    def rbody(v, _):
        acc = dacc[pl.ds(v * 16, 16)]
        for k in range(1, _SLOTS):
            acc = acc + dacc[pl.ds(k * N + v * 16, 16)]
        dacc[pl.ds(v * 16, 16)] = acc
        return 0

    lax.fori_loop(0, N // 16, rbody, 0)
    pltpu.sync_copy(ex_v, ex_out.at[pl.ds(base, _EPW)])
    pltpu.sync_copy(dacc.at[pl.ds(0, N)], denp_out.at[pl.ds(w * N, N)])


def _attn_ex(sq, sk, srcf, dstf, bv):
    f = pl.kernel(
        _attn_ex_body,
        compiler_params=pltpu.CompilerParams(needs_layout_passes=False),
        out_type=(jax.ShapeDtypeStruct((E,), jnp.float32),
                  jax.ShapeDtypeStruct((_NW * N,), jnp.float32)),
        mesh=plsc.VectorSubcoreMesh(core_axis_name="c", subcore_axis_name="s"),
        scratch_types=[
            pltpu.VMEM((N,), jnp.float32),
            pltpu.VMEM((M,), jnp.float32),
            pltpu.VMEM((_EPW,), jnp.int32),
            pltpu.VMEM((_EPW,), jnp.int32),
            pltpu.VMEM((_EPW,), jnp.float32),
            pltpu.VMEM((16,), jnp.float32),
            pltpu.VMEM((_SLOTS * N,), jnp.float32),
        ],
    )
    return f(sq, sk, srcf, dstf, bv)


_HN_BLK = 16


def _hn_body(ktab, src2d, dst2d, ex2d, rec, zeros, out,
             src_v, dst_v, ex_v, rec_v, al_v, rows0, rows1, acc, gsem0, gsem1):
    c = lax.axis_index("c")
    s = lax.axis_index("s")
    w = s * _NC + c

    rows_per_sub = 624
    pltpu.sync_copy(zeros.at[pl.ds(s * rows_per_sub, rows_per_sub)],
                    acc.at[pl.ds(s * rows_per_sub, rows_per_sub)])

    @pl.when(s == 0)
    def _():
        pltpu.sync_copy(zeros.at[pl.ds(_NS * rows_per_sub, N - _NS * rows_per_sub)],
                        acc.at[pl.ds(_NS * rows_per_sub, N - _NS * rows_per_sub)])

    pltpu.sync_copy(rec, rec_v)
    plsc.subcore_barrier()

    rows = (rows0, rows1)
    sems = (gsem0, gsem1)
    descs = [None, None]

    def start(j, b):
        descs[b] = pltpu.async_copy(ktab.at[dst_v.at[j]], rows[b], sems[b])

    offs = [v * 16 for v in range(7)] + [_CHUNK - 16]
    for blk in range(_ROWS_PER_W // _HN_BLK):
        rbase = w * _ROWS_PER_W + blk * _HN_BLK
        pltpu.sync_copy(src2d.at[pl.ds(rbase, _HN_BLK)], src_v)
        pltpu.sync_copy(dst2d.at[pl.ds(rbase, _HN_BLK)], dst_v)
        pltpu.sync_copy(ex2d.at[pl.ds(rbase, _HN_BLK)], ex_v)
        for j in range(_HN_BLK):
            for off in offs:
                s16 = src_v[j, pl.ds(off, 16)]
                e16 = ex_v[j, pl.ds(off, 16)]
                al_v[pl.ds(j * _CHUNK + off, 16)] = \
                    e16 * plsc.load_gather(rec_v, [s16])
        start(0, 0)
        for j in range(_HN_BLK):
            b = j & 1
            if j + 1 < _HN_BLK:
                start(j + 1, (j + 1) & 1)
            descs[b].wait()

            def ebody(i, _):
                sp = plsc.load_gather(al_v, [jnp.full((16,), j * _CHUNK, jnp.int32) + i])
                for t in range(8):
                    rows[b][i, pl.ds(t * 16, 16)] = rows[b][i, pl.ds(t * 16, 16)] * sp
                return 0

            lax.fori_loop(0, _CHUNK, ebody, 0)
            pltpu.sync_copy(rows[b], acc.at[src_v.at[j]], add=True)

    plsc.subcore_barrier()
    pltpu.sync_copy(acc.at[pl.ds(s * rows_per_sub, rows_per_sub)],
                    out.at[c].at[pl.ds(s * rows_per_sub, rows_per_sub)])

    @pl.when(s == 0)
    def _():
        pltpu.sync_copy(acc.at[pl.ds(_NS * rows_per_sub, N - _NS * rows_per_sub)],
                        out.at[c].at[pl.ds(_NS * rows_per_sub, N - _NS * rows_per_sub)])


def _hn_pass(ktab, src2d, dst2d, ex2d, rec, zeros):
    f = pl.kernel(
        _hn_body,
        compiler_params=pltpu.CompilerParams(needs_layout_passes=False),
        out_type=jax.ShapeDtypeStruct((_NC, N, H), jnp.float32),
        mesh=plsc.VectorSubcoreMesh(core_axis_name="c", subcore_axis_name="s"),
        scratch_types=[
            pltpu.VMEM((_HN_BLK, _CHUNK), jnp.int32),
            pltpu.VMEM((_HN_BLK, _CHUNK), jnp.int32),
            pltpu.VMEM((_HN_BLK, _CHUNK), jnp.float32),
            pltpu.VMEM((N,), jnp.float32),
            pltpu.VMEM((_HN_BLK * _CHUNK,), jnp.float32),
            pltpu.VMEM((_CHUNK, H), jnp.float32),
            pltpu.VMEM((_CHUNK, H), jnp.float32),
            pltpu.VMEM_SHARED((N, H), jnp.float32),
            pltpu.SemaphoreType.DMA,
            pltpu.SemaphoreType.DMA,
        ],
    )
    return f(ktab, src2d, dst2d, ex2d, rec, zeros)


_MIN_OWN = 312
_MIN_ACC = 328
_SC_BLK = 4000
_MCAP = 4224


def _min_body(hn, srcf, dstf, out, sb0, db0, sb1, db1, msrc, mdlo,
              rows0, rows1, acc, ssem0, ssem1, gsem0, gsem1):
    c = lax.axis_index("c")
    s = lax.axis_index("s")
    w = s * _NC + c
    lo = w * _MIN_OWN
    hi = jnp.where(w == _NW - 1, M, lo + _MIN_OWN)
    lo16 = jnp.broadcast_to(lo, (16,))
    hi16 = jnp.broadcast_to(hi, (16,))
    sbufs = (sb0, sb1)
    dbufs = (db0, db1)
    ssems = (ssem0, ssem1)
    rbufs = (rows0, rows1)
    gsems = (gsem0, gsem1)

    def zb(v, _):
        msrc[pl.ds(v * 16, 16)] = jnp.zeros((16,), jnp.int32)
        return 0

    lax.fori_loop(0, _MCAP // 16, zb, 0)

    inf16 = jnp.full((16,), jnp.inf, jnp.float32)

    def accinit(v, _):
        acc[v // 8, pl.ds((v % 8) * 16, 16)] = inf16
        return 0

    lax.fori_loop(0, _MIN_ACC * 8, accinit, 0)

    nblk = E // _SC_BLK
    iota16 = lax.iota(jnp.int32, 16)

    def stage_start(blk, b):
        pltpu.async_copy(srcf.at[pl.ds(blk * _SC_BLK, _SC_BLK)], sbufs[b], ssems[b])
        pltpu.async_copy(dstf.at[pl.ds(blk * _SC_BLK, _SC_BLK)], dbufs[b], ssems[b])

    def stage_wait(b):
        pltpu.make_async_copy(srcf.at[pl.ds(0, _SC_BLK)], sbufs[b], ssems[b]).wait()
        pltpu.make_async_copy(dstf.at[pl.ds(0, _SC_BLK)], dbufs[b], ssems[b]).wait()

    def gather_start(cidx, rb):
        pltpu.async_copy(hn.at[msrc.at[pl.ds(cidx * 128, 128)]],
                         rbufs[rb], gsems[rb])

    def gather_wait(rb):
        pltpu.make_async_copy(hn.at[msrc.at[pl.ds(0, 128)]],
                              rbufs[rb], gsems[rb]).wait()

    def rmw_edge(cidx, i, rref):
        dl = plsc.load_gather(mdlo, [jnp.full((16,), 0, jnp.int32)
                                     + (cidx * 128 + i)])
        for t in range(8):
            col = iota16 + (t * 16)
            cur = plsc.load_gather(acc, [dl, col])
            r = rref[i, pl.ds(t * 16, 16)]
            plsc.store_scatter(acc, [dl, col], jnp.minimum(cur, r))

    def rmw_chunk(cidx, rb, cnt):
        nb = jnp.minimum(cnt - cidx * 128, 128)

        def e2(i, _):
            rmw_edge(cidx, 2 * i, rbufs[rb])
            rmw_edge(cidx, 2 * i + 1, rbufs[rb])
            return 0

        lax.fori_loop(0, nb // 2, e2, 0)

        @pl.when(nb % 2 == 1)
        def _():
            rmw_edge(cidx, nb - 1, rbufs[rb])

    def process(cnt):
        nch = (cnt + 127) // 128

        @pl.when(nch > 0)
        def _():
            gather_start(0, 0)

        def c2(p2, _):
            c0 = 2 * p2
            c1 = c0 + 1

            @pl.when(c0 < nch)
            def _():
                @pl.when(c1 < nch)
                def _():
                    gather_start(c1, 1)
                gather_wait(0)
                rmw_chunk(c0, 0, cnt)

                @pl.when(c0 + 2 < nch)
                def _():
                    gather_start(c0 + 2, 0)

            @pl.when(c1 < nch)
            def _():
                gather_wait(1)
                rmw_chunk(c1, 1, cnt)

                @pl.when(c1 + 2 < nch)
                def _():
                    gather_start(c1 + 2, 1)

            return 0

        lax.fori_loop(0, (nch + 1) // 2, c2, 0)

    def scan(b):
        def scan_body(v, cnt):
            s16 = sbufs[b][pl.ds(v * 16, 16)]
            d16 = dbufs[b][pl.ds(v * 16, 16)]
            m = jnp.logical_and(d16 >= lo16, d16 < hi16)
            plsc.store_compressed(msrc.at[pl.ds(cnt, 16)], s16, mask=m)
            plsc.store_compressed(mdlo.at[pl.ds(cnt, 16)], d16 - lo16, mask=m)
            return cnt + jnp.sum(m.astype(jnp.int32))

        return lax.fori_loop(0, _SC_BLK // 16, scan_body, 0)

    stage_start(0, 0)

    def blk2(p, _):
        b0 = 2 * p
        stage_start(b0 + 1, 1)
        stage_wait(0)
        process(scan(0))

        @pl.when(b0 + 2 < nblk)
        def _():
            stage_start(b0 + 2, 0)

        stage_wait(1)
        process(scan(1))
        return 0

    lax.fori_loop(0, nblk // 2, blk2, 0)

    def fin(v, _):
        r = v // 8
        off = (v % 8) * 16
        val = acc[r, pl.ds(off, 16)]
        acc[r, pl.ds(off, 16)] = jnp.where(val * 0.0 == 0.0, val, 0.0)
        return 0

    lax.fori_loop(0, _MIN_ACC * 8, fin, 0)
    pltpu.sync_copy(acc.at[pl.ds(0, _MIN_OWN)], out.at[pl.ds(lo, _MIN_OWN)])

    @pl.when(w == _NW - 1)
    def _():
        pltpu.sync_copy(acc.at[pl.ds(_MIN_OWN, _MIN_ACC - _MIN_OWN)],
                        out.at[pl.ds(M - (_MIN_ACC - _MIN_OWN), _MIN_ACC - _MIN_OWN)])


def _min_pass(hn, srcf, dstf):
    f = pl.kernel(
        _min_body,
        compiler_params=pltpu.CompilerParams(needs_layout_passes=False),
        out_type=jax.ShapeDtypeStruct((M, H), jnp.float32),
        mesh=plsc.VectorSubcoreMesh(core_axis_name="c", subcore_axis_name="s"),
        scratch_types=[
            pltpu.VMEM((_SC_BLK,), jnp.int32),
            pltpu.VMEM((_SC_BLK,), jnp.int32),
            pltpu.VMEM((_SC_BLK,), jnp.int32),
            pltpu.VMEM((_SC_BLK,), jnp.int32),
            pltpu.VMEM((_MCAP,), jnp.int32),
            pltpu.VMEM((_MCAP,), jnp.int32),
            pltpu.VMEM((128, H), jnp.float32),
            pltpu.VMEM((128, H), jnp.float32),
            pltpu.VMEM((_MIN_ACC, H), jnp.float32),
            pltpu.SemaphoreType.DMA,
            pltpu.SemaphoreType.DMA,
            pltpu.SemaphoreType.DMA,
            pltpu.SemaphoreType.DMA,
        ],
    )
    return f(hn, srcf, dstf)


@functools.partial(jax.jit, static_argnums=())
def _seg_sum(table, gidx_rows, sidx_rows, zeros):
    f = pl.kernel(
        _seg_sum_body,
        out_type=jax.ShapeDtypeStruct((_NC, M, H), jnp.float32),
        mesh=plsc.VectorSubcoreMesh(core_axis_name="c", subcore_axis_name="s"),
        scratch_types=[
            pltpu.VMEM((_IDX_BLOCK, _CHUNK), jnp.int32),
            pltpu.VMEM((_IDX_BLOCK, _CHUNK), jnp.int32),
            pltpu.VMEM((_CHUNK, H), jnp.float32),
            pltpu.VMEM((_CHUNK, H), jnp.float32),
            pltpu.VMEM_SHARED((M, H), jnp.float32),
            pltpu.SemaphoreType.DMA,
            pltpu.SemaphoreType.DMA,
        ],
    )
    return f(table, gidx_rows, sidx_rows, zeros)


def kernel(x, x_struct, x_e, edge_index, W1e, b1e, W2e, b2e, W1n, b1n, W2n, b2n, Wq, bq, Wk, bk, att, g1, be1, a1, Wf, bf, g2, be2, a2, Wc1, bc1, Wc2, bc2):
    src = edge_index[0]
    dst = edge_index[1]
    src_rows = src.reshape(E // _CHUNK, _CHUNK)
    dst_rows = dst.reshape(E // _CHUNK, _CHUNK)
    zeros_mh = jnp.zeros((M, H), jnp.float32)

    xe = _lrelu(x_e @ W1e + b1e)
    xe = _lrelu(xe @ W2e + b2e)
    h = _lrelu(x @ W1n + b1n)
    ones = jnp.ones((E,), jnp.float32)
    deg_e = jnp.maximum(jax.ops.segment_sum(ones, dst, num_segments=M), 1.0)
    p = _seg_sum(h, src_rows, dst_rows, zeros_mh)
    e_agg = (p[0] + p[1]) / deg_e[:, None]
    deg_n = jnp.maximum(jax.ops.segment_sum(ones, src, num_segments=N), 1.0)
    p = _seg_sum(e_agg, dst_rows, src_rows, zeros_mh)
    n_agg = (p[0] + p[1]) / deg_n[:, None]
    h = _lrelu((h + n_agg) @ W2n + b2n)
    q = h @ Wq + bq
    k = xe @ Wk + bk
    sq = q @ att[:H]
    sk = k @ att[H:]
    bscal = _lrelu(jnp.max(sq) + jnp.max(sk))
    bv = jnp.broadcast_to(bscal, (16,))
    ex, denp = _attn_ex(sq, sk, src, dst, bv)
    denom = jnp.sum(denp.reshape(_NW, N), axis=0)
    rec = 1.0 / (denom + 1e-16)
    ex_rows = ex.reshape(E // _CHUNK, _CHUNK)
    p = _hn_pass(k, src_rows, dst_rows, ex_rows, rec, zeros_mh)
    hn = p[0] + p[1]
    hm = _min_pass(hn, src, dst)
    z = jnp.concatenate([hm, xe], axis=1)
    z = _graph_norm(z, g1, be1, a1)
    z = _lrelu(z @ Wf + bf)
    z = _graph_norm(z, g2, be2, a2)
    z = _lrelu(z)
    z = _lrelu(z @ Wc1 + bc1)
    z = z @ Wc2 + bc2
    return z


# speedup vs baseline: 2.5575x; 2.5575x over previous
import functools

import jax
import jax.numpy as jnp
from jax import lax
from jax.experimental import pallas as pl
from jax.experimental.pallas import tpu as pltpu
from jax.experimental.pallas import tpu_sc as plsc

N = 10000
M = 10000
E = 320000
H = 128

_NC = 2
_NS = 16
_NW = _NC * _NS
_CHUNK = 125
_ROWS_PER_W = E // _CHUNK // _NW
_IDX_BLOCK = 40


def _lrelu(v):
    return jnp.where(v >= 0, v, 0.01 * v)


def _graph_norm(x, gamma, beta, alpha, eps=1e-5):
    mean = jnp.mean(x, axis=0, keepdims=True)
    xc = x - alpha * mean
    var = jnp.mean(xc * xc, axis=0, keepdims=True)
    return gamma * xc / jnp.sqrt(var + eps) + beta


def _seg_sum_body(table, gidx, sidx, zeros, out, gidx_v, sidx_v, rows0, rows1,
                  acc, gsem0, gsem1):
    c = lax.axis_index("c")
    s = lax.axis_index("s")
    w = s * _NC + c

    rows_per_sub = 624
    pltpu.sync_copy(zeros.at[pl.ds(s * rows_per_sub, rows_per_sub)],
                    acc.at[pl.ds(s * rows_per_sub, rows_per_sub)])

    @pl.when(s == 0)
    def _():
        pltpu.sync_copy(zeros.at[pl.ds(_NS * rows_per_sub, M - _NS * rows_per_sub)],
                        acc.at[pl.ds(_NS * rows_per_sub, M - _NS * rows_per_sub)])
    plsc.subcore_barrier()

    rows = (rows0, rows1)
    sems = (gsem0, gsem1)
    descs = [None, None]

    def start(j, b):
        descs[b] = pltpu.async_copy(table.at[gidx_v.at[j]], rows[b], sems[b])

    for blk in range(_ROWS_PER_W // _IDX_BLOCK):
        base = w * _ROWS_PER_W + blk * _IDX_BLOCK
        pltpu.sync_copy(gidx.at[pl.ds(base, _IDX_BLOCK)], gidx_v)
        pltpu.sync_copy(sidx.at[pl.ds(base, _IDX_BLOCK)], sidx_v)
        start(0, 0)
        for j in range(_IDX_BLOCK):
            b = j & 1
            if j + 1 < _IDX_BLOCK:
                start(j + 1, (j + 1) & 1)
            descs[b].wait()
            pltpu.sync_copy(rows[b], acc.at[sidx_v.at[j]], add=True)

    plsc.subcore_barrier()
    pltpu.sync_copy(acc.at[pl.ds(s * rows_per_sub, rows_per_sub)],
                    out.at[c].at[pl.ds(s * rows_per_sub, rows_per_sub)])

    @pl.when(s == 0)
    def _():
        pltpu.sync_copy(acc.at[pl.ds(_NS * rows_per_sub, M - _NS * rows_per_sub)],
                        out.at[c].at[pl.ds(_NS * rows_per_sub, M - _NS * rows_per_sub)])


_EPW = E // _NW
_NV = _EPW // 16
_SLOTS = 8


def _attn_ex_body(sq, sk, srcf, dstf, bv, ex_out, denp_out,
                  sq_v, sk_v, src_v, dst_v, ex_v, b_v, dacc):
    c = lax.axis_index("c")
    s = lax.axis_index("s")
    w = s * _NC + c
    base = w * _EPW
    pltpu.sync_copy(sq, sq_v)
    pltpu.sync_copy(sk, sk_v)
    pltpu.sync_copy(srcf.at[pl.ds(base, _EPW)], src_v)
    pltpu.sync_copy(dstf.at[pl.ds(base, _EPW)], dst_v)
    pltpu.sync_copy(bv, b_v)
    b16 = b_v[...]
    iota = lax.iota(jnp.int32, 16)
    mlo = iota < _SLOTS
    mhi = jnp.logical_not(mlo)
    slot_off = (iota % _SLOTS) * N

    def zbody(v, _):
        dacc[pl.ds(v * 16, 16)] = jnp.zeros((16,), jnp.float32)
        return 0

    lax.fori_loop(0, _SLOTS * N // 16, zbody, 0)

    def body(v, _):
        s16 = src_v[pl.ds(v * 16, 16)]
        d16 = dst_v[pl.ds(v * 16, 16)]
        a = plsc.load_gather(sq_v, [s16])
        b = plsc.load_gather(sk_v, [d16])
        sc = a + b
        sc = jnp.where(sc >= 0, sc, 0.01 * sc)
        e = jnp.exp(sc - b16)
        ex_v[pl.ds(v * 16, 16)] = e
        didx = slot_off + s16
        plsc.addupdate_scatter(dacc, [didx], e, mask=mlo)
        plsc.addupdate_scatter(dacc, [didx], e, mask=mhi)
        return 0

    lax.fori_loop(0, _NV, body, 0)

    def rbody(v, _):
        acc = dacc[pl.ds(v * 16, 16)]
        for k in range(1, _SLOTS):
            acc = acc + dacc[pl.ds(k * N + v * 16, 16)]
        dacc[pl.ds(v * 16, 16)] = acc
        return 0

    lax.fori_loop(0, N // 16, rbody, 0)
    pltpu.sync_copy(ex_v, ex_out.at[pl.ds(base, _EPW)])
    pltpu.sync_copy(dacc.at[pl.ds(0, N)], denp_out.at[pl.ds(w * N, N)])


def _attn_ex(sq, sk, srcf, dstf, bv):
    f = pl.kernel(
        _attn_ex_body,
        compiler_params=pltpu.CompilerParams(needs_layout_passes=False),
        out_type=(jax.ShapeDtypeStruct((E,), jnp.float32),
                  jax.ShapeDtypeStruct((_NW * N,), jnp.float32)),
        mesh=plsc.VectorSubcoreMesh(core_axis_name="c", subcore_axis_name="s"),
        scratch_types=[
            pltpu.VMEM((N,), jnp.float32),
            pltpu.VMEM((M,), jnp.float32),
            pltpu.VMEM((_EPW,), jnp.int32),
            pltpu.VMEM((_EPW,), jnp.int32),
            pltpu.VMEM((_EPW,), jnp.float32),
            pltpu.VMEM((16,), jnp.float32),
            pltpu.VMEM((_SLOTS * N,), jnp.float32),
        ],
    )
    return f(sq, sk, srcf, dstf, bv)


_HN_BLK = 16


def _hn_body(ktab, src2d, dst2d, ex2d, rec, zeros, out,
             src_v, dst_v, ex_v, rec_v, al_v, rows0, rows1, acc, gsem0, gsem1):
    c = lax.axis_index("c")
    s = lax.axis_index("s")
    w = s * _NC + c

    rows_per_sub = 624
    pltpu.sync_copy(zeros.at[pl.ds(s * rows_per_sub, rows_per_sub)],
                    acc.at[pl.ds(s * rows_per_sub, rows_per_sub)])

    @pl.when(s == 0)
    def _():
        pltpu.sync_copy(zeros.at[pl.ds(_NS * rows_per_sub, N - _NS * rows_per_sub)],
                        acc.at[pl.ds(_NS * rows_per_sub, N - _NS * rows_per_sub)])

    pltpu.sync_copy(rec, rec_v)
    plsc.subcore_barrier()

    rows = (rows0, rows1)
    sems = (gsem0, gsem1)
    descs = [None, None]

    def start(j, b):
        descs[b] = pltpu.async_copy(ktab.at[dst_v.at[j]], rows[b], sems[b])

    offs = [v * 16 for v in range(7)] + [_CHUNK - 16]
    for blk in range(_ROWS_PER_W // _HN_BLK):
        rbase = w * _ROWS_PER_W + blk * _HN_BLK
        pltpu.sync_copy(src2d.at[pl.ds(rbase, _HN_BLK)], src_v)
        pltpu.sync_copy(dst2d.at[pl.ds(rbase, _HN_BLK)], dst_v)
        pltpu.sync_copy(ex2d.at[pl.ds(rbase, _HN_BLK)], ex_v)
        for j in range(_HN_BLK):
            for off in offs:
                s16 = src_v[j, pl.ds(off, 16)]
                e16 = ex_v[j, pl.ds(off, 16)]
                al_v[pl.ds(j * _CHUNK + off, 16)] = \
                    e16 * plsc.load_gather(rec_v, [s16])
        start(0, 0)
        for j in range(_HN_BLK):
            b = j & 1
            if j + 1 < _HN_BLK:
                start(j + 1, (j + 1) & 1)
            descs[b].wait()

            def ebody(i, _):
                sp = plsc.load_gather(al_v, [jnp.full((16,), j * _CHUNK, jnp.int32) + i])
                for t in range(8):
                    rows[b][i, pl.ds(t * 16, 16)] = rows[b][i, pl.ds(t * 16, 16)] * sp
                return 0

            lax.fori_loop(0, _CHUNK, ebody, 0)
            pltpu.sync_copy(rows[b], acc.at[src_v.at[j]], add=True)

    plsc.subcore_barrier()
    pltpu.sync_copy(acc.at[pl.ds(s * rows_per_sub, rows_per_sub)],
                    out.at[c].at[pl.ds(s * rows_per_sub, rows_per_sub)])

    @pl.when(s == 0)
    def _():
        pltpu.sync_copy(acc.at[pl.ds(_NS * rows_per_sub, N - _NS * rows_per_sub)],
                        out.at[c].at[pl.ds(_NS * rows_per_sub, N - _NS * rows_per_sub)])


def _hn_pass(ktab, src2d, dst2d, ex2d, rec, zeros):
    f = pl.kernel(
        _hn_body,
        compiler_params=pltpu.CompilerParams(needs_layout_passes=False),
        out_type=jax.ShapeDtypeStruct((_NC, N, H), jnp.float32),
        mesh=plsc.VectorSubcoreMesh(core_axis_name="c", subcore_axis_name="s"),
        scratch_types=[
            pltpu.VMEM((_HN_BLK, _CHUNK), jnp.int32),
            pltpu.VMEM((_HN_BLK, _CHUNK), jnp.int32),
            pltpu.VMEM((_HN_BLK, _CHUNK), jnp.float32),
            pltpu.VMEM((N,), jnp.float32),
            pltpu.VMEM((_HN_BLK * _CHUNK,), jnp.float32),
            pltpu.VMEM((_CHUNK, H), jnp.float32),
            pltpu.VMEM((_CHUNK, H), jnp.float32),
            pltpu.VMEM_SHARED((N, H), jnp.float32),
            pltpu.SemaphoreType.DMA,
            pltpu.SemaphoreType.DMA,
        ],
    )
    return f(ktab, src2d, dst2d, ex2d, rec, zeros)


_MIN_OWN = 312
_MIN_ACC = 328
_SC_BLK = 4000
_MCAP = 8192


def _min_body(hn, srcf, dstf, out, sb0, db0, sb1, db1, msrc, mdlo,
              rows0, rows1, acc, ssem0, ssem1, gsem0, gsem1):
    c = lax.axis_index("c")
    s = lax.axis_index("s")
    w = s * _NC + c
    lo = w * _MIN_OWN
    hi = jnp.where(w == _NW - 1, M, lo + _MIN_OWN)
    lo16 = jnp.broadcast_to(lo, (16,))
    hi16 = jnp.broadcast_to(hi, (16,))
    sbufs = (sb0, sb1)
    dbufs = (db0, db1)
    ssems = (ssem0, ssem1)
    rbufs = (rows0, rows1)
    gsems = (gsem0, gsem1)

    def zb(v, _):
        msrc[pl.ds(v * 16, 16)] = jnp.zeros((16,), jnp.int32)
        return 0

    lax.fori_loop(0, _MCAP // 16, zb, 0)

    inf16 = jnp.full((16,), jnp.inf, jnp.float32)

    def accinit(v, _):
        acc[v // 8, pl.ds((v % 8) * 16, 16)] = inf16
        return 0

    lax.fori_loop(0, _MIN_ACC * 8, accinit, 0)

    nblk = E // _SC_BLK
    iota16 = lax.iota(jnp.int32, 16)

    def stage_start(blk, b):
        pltpu.async_copy(srcf.at[pl.ds(blk * _SC_BLK, _SC_BLK)], sbufs[b], ssems[b])
        pltpu.async_copy(dstf.at[pl.ds(blk * _SC_BLK, _SC_BLK)], dbufs[b], ssems[b])

    def stage_wait(b):
        pltpu.make_async_copy(srcf.at[pl.ds(0, _SC_BLK)], sbufs[b], ssems[b]).wait()
        pltpu.make_async_copy(dstf.at[pl.ds(0, _SC_BLK)], dbufs[b], ssems[b]).wait()

    def gather_start(cidx, rb):
        pltpu.async_copy(hn.at[msrc.at[pl.ds(cidx * 128, 128)]],
                         rbufs[rb], gsems[rb])

    def gather_wait(rb):
        pltpu.make_async_copy(hn.at[msrc.at[pl.ds(0, 128)]],
                              rbufs[rb], gsems[rb]).wait()

    def rmw_edge(cidx, i, rref):
        dl = plsc.load_gather(mdlo, [jnp.full((16,), 0, jnp.int32)
                                     + (cidx * 128 + i)])
        for t in range(8):
            col = iota16 + (t * 16)
            cur = plsc.load_gather(acc, [dl, col])
            r = rref[i, pl.ds(t * 16, 16)]
            plsc.store_scatter(acc, [dl, col], jnp.minimum(cur, r))

    def rmw_full(cidx, rb):
        def e4(i, _):
            for k in range(4):
                rmw_edge(cidx, 4 * i + k, rbufs[rb])
            return 0

        lax.fori_loop(0, 32, e4, 0)

    def process_full(cnt):
        nch = cnt // 128
        gather_start(0, 0)

        def c2(p2, _):
            c0 = 2 * p2
            c1 = c0 + 1
            gather_start(c1, 1)
            gather_wait(0)
            rmw_full(c0, 0)
            gather_start(c0 + 2, 0)

            gather_wait(1)

            @pl.when(c1 < nch)
            def _():
                rmw_full(c1, 1)

            return 0

        lax.fori_loop(0, (nch + 1) // 2, c2, 0)
        gather_wait(0)
        for k in range(8):
            msrc[pl.ds(k * 16, 16)] = msrc[pl.ds(nch * 128 + k * 16, 16)]
            mdlo[pl.ds(k * 16, 16)] = mdlo[pl.ds(nch * 128 + k * 16, 16)]
        return cnt - nch * 128

    def scan(b, cnt0):
        def scan_body(v, cnt):
            s16 = sbufs[b][pl.ds(v * 16, 16)]
            d16 = dbufs[b][pl.ds(v * 16, 16)]
            m = jnp.logical_and(d16 >= lo16, d16 < hi16)
            plsc.store_compressed(msrc.at[pl.ds(cnt, 16)], s16, mask=m)
            plsc.store_compressed(mdlo.at[pl.ds(cnt, 16)], d16 - lo16, mask=m)
            return cnt + jnp.sum(m.astype(jnp.int32))

        return lax.fori_loop(0, _SC_BLK // 16, scan_body, cnt0)

    stage_start(0, 0)
    _FLUSH = _MCAP - _SC_BLK

    def maybe_flush(cnt):
        return lax.cond(cnt >= _FLUSH, process_full, lambda c: c, cnt)

    def blk2(p, cnt):
        b0 = 2 * p
        stage_start(b0 + 1, 1)
        stage_wait(0)
        cnt = maybe_flush(scan(0, cnt))

        @pl.when(b0 + 2 < nblk)
        def _():
            stage_start(b0 + 2, 0)

        stage_wait(1)
        cnt = maybe_flush(scan(1, cnt))
        return cnt

    cnt = lax.fori_loop(0, nblk // 2, blk2, 0)

    def fchunk(cidx, cnt):
        pltpu.async_copy(hn.at[msrc.at[pl.ds(cidx * 128, 128)]],
                         rows0, gsem0).wait()
        nb = jnp.minimum(cnt - cidx * 128, 128)

        def e1(i, _):
            rmw_edge(cidx, i, rows0)
            return 0

        lax.fori_loop(0, nb, e1, 0)
        return cnt

    lax.fori_loop(0, (cnt + 127) // 128, fchunk, cnt)

    def fin(v, _):
        r = v // 8
        off = (v % 8) * 16
        val = acc[r, pl.ds(off, 16)]
        acc[r, pl.ds(off, 16)] = jnp.where(val * 0.0 == 0.0, val, 0.0)
        return 0

    lax.fori_loop(0, _MIN_ACC * 8, fin, 0)
    pltpu.sync_copy(acc.at[pl.ds(0, _MIN_OWN)], out.at[pl.ds(lo, _MIN_OWN)])

    @pl.when(w == _NW - 1)
    def _():
        pltpu.sync_copy(acc.at[pl.ds(_MIN_OWN, _MIN_ACC - _MIN_OWN)],
                        out.at[pl.ds(M - (_MIN_ACC - _MIN_OWN), _MIN_ACC - _MIN_OWN)])


def _min_pass(hn, srcf, dstf):
    f = pl.kernel(
        _min_body,
        compiler_params=pltpu.CompilerParams(needs_layout_passes=False),
        out_type=jax.ShapeDtypeStruct((M, H), jnp.float32),
        mesh=plsc.VectorSubcoreMesh(core_axis_name="c", subcore_axis_name="s"),
        scratch_types=[
            pltpu.VMEM((_SC_BLK,), jnp.int32),
            pltpu.VMEM((_SC_BLK,), jnp.int32),
            pltpu.VMEM((_SC_BLK,), jnp.int32),
            pltpu.VMEM((_SC_BLK,), jnp.int32),
            pltpu.VMEM((_MCAP,), jnp.int32),
            pltpu.VMEM((_MCAP,), jnp.int32),
            pltpu.VMEM((128, H), jnp.float32),
            pltpu.VMEM((128, H), jnp.float32),
            pltpu.VMEM((_MIN_ACC, H), jnp.float32),
            pltpu.SemaphoreType.DMA,
            pltpu.SemaphoreType.DMA,
            pltpu.SemaphoreType.DMA,
            pltpu.SemaphoreType.DMA,
        ],
    )
    return f(hn, srcf, dstf)


@functools.partial(jax.jit, static_argnums=())
def _seg_sum(table, gidx_rows, sidx_rows, zeros):
    f = pl.kernel(
        _seg_sum_body,
        out_type=jax.ShapeDtypeStruct((_NC, M, H), jnp.float32),
        mesh=plsc.VectorSubcoreMesh(core_axis_name="c", subcore_axis_name="s"),
        scratch_types=[
            pltpu.VMEM((_IDX_BLOCK, _CHUNK), jnp.int32),
            pltpu.VMEM((_IDX_BLOCK, _CHUNK), jnp.int32),
            pltpu.VMEM((_CHUNK, H), jnp.float32),
            pltpu.VMEM((_CHUNK, H), jnp.float32),
            pltpu.VMEM_SHARED((M, H), jnp.float32),
            pltpu.SemaphoreType.DMA,
            pltpu.SemaphoreType.DMA,
        ],
    )
    return f(table, gidx_rows, sidx_rows, zeros)


def kernel(x, x_struct, x_e, edge_index, W1e, b1e, W2e, b2e, W1n, b1n, W2n, b2n, Wq, bq, Wk, bk, att, g1, be1, a1, Wf, bf, g2, be2, a2, Wc1, bc1, Wc2, bc2):
    src = edge_index[0]
    dst = edge_index[1]
    src_rows = src.reshape(E // _CHUNK, _CHUNK)
    dst_rows = dst.reshape(E // _CHUNK, _CHUNK)
    zeros_mh = jnp.zeros((M, H), jnp.float32)

    xe = _lrelu(x_e @ W1e + b1e)
    xe = _lrelu(xe @ W2e + b2e)
    h = _lrelu(x @ W1n + b1n)
    ones = jnp.ones((E,), jnp.float32)
    deg_e = jnp.maximum(jax.ops.segment_sum(ones, dst, num_segments=M), 1.0)
    p = _seg_sum(h, src_rows, dst_rows, zeros_mh)
    e_agg = (p[0] + p[1]) / deg_e[:, None]
    deg_n = jnp.maximum(jax.ops.segment_sum(ones, src, num_segments=N), 1.0)
    p = _seg_sum(e_agg, dst_rows, src_rows, zeros_mh)
    n_agg = (p[0] + p[1]) / deg_n[:, None]
    h = _lrelu((h + n_agg) @ W2n + b2n)
    q = h @ Wq + bq
    k = xe @ Wk + bk
    sq = q @ att[:H]
    sk = k @ att[H:]
    bscal = _lrelu(jnp.max(sq) + jnp.max(sk))
    bv = jnp.broadcast_to(bscal, (16,))
    ex, denp = _attn_ex(sq, sk, src, dst, bv)
    denom = jnp.sum(denp.reshape(_NW, N), axis=0)
    rec = 1.0 / (denom + 1e-16)
    ex_rows = ex.reshape(E // _CHUNK, _CHUNK)
    p = _hn_pass(k, src_rows, dst_rows, ex_rows, rec, zeros_mh)
    hn = p[0] + p[1]
    hm = _min_pass(hn, src, dst)
    z = jnp.concatenate([hm, xe], axis=1)
    z = _graph_norm(z, g1, be1, a1)
    z = _lrelu(z @ Wf + bf)
    z = _graph_norm(z, g2, be2, a2)
    z = _lrelu(z)
    z = _lrelu(z @ Wc1 + bc1)
    z = z @ Wc2 + bc2
    return z


# speedup vs baseline: 3.1671x; 1.2384x over previous
import functools

import jax
import jax.numpy as jnp
from jax import lax
from jax.experimental import pallas as pl
from jax.experimental.pallas import tpu as pltpu
from jax.experimental.pallas import tpu_sc as plsc

N = 10000
M = 10000
E = 320000
H = 128

_NC = 2
_NS = 16
_NW = _NC * _NS
_CHUNK = 125
_ROWS_PER_W = E // _CHUNK // _NW
_IDX_BLOCK = 40


def _lrelu(v):
    return jnp.where(v >= 0, v, 0.01 * v)


_RB = 2000


def _rb_spec():
    return pl.BlockSpec((_RB, H), lambda i: (i, 0))


def _full(shape):
    nd = len(shape)
    return pl.BlockSpec(shape, lambda i: (0,) * nd)


def _tc_pre(x_e, x, W1e, b1e, W2e, b2e, W1n, b1n):
    def body(xe_ref, x_ref, w1e, bb1e, w2e, bb2e, w1n, bb1n, oxe, oh1):
        t = _lrelu(jnp.dot(xe_ref[...], w1e[...],
                           preferred_element_type=jnp.float32) + bb1e[...])
        oxe[...] = _lrelu(jnp.dot(t, w2e[...],
                                  preferred_element_type=jnp.float32) + bb2e[...])
        oh1[...] = _lrelu(jnp.dot(x_ref[...], w1n[...],
                                  preferred_element_type=jnp.float32) + bb1n[...])

    return pl.pallas_call(
        body,
        grid=(M // _RB,),
        in_specs=[_rb_spec(), _rb_spec(), _full((H, H)), _full((H,)),
                  _full((H, H)), _full((H,)), _full((H, H)), _full((H,))],
        out_specs=[_rb_spec(), _rb_spec()],
        out_shape=[jax.ShapeDtypeStruct((M, H), jnp.float32),
                   jax.ShapeDtypeStruct((N, H), jnp.float32)],
    )(x_e, x, W1e, b1e, W2e, b2e, W1n, b1n)


def _tc_agg_div(p, degp):
    def body(p_ref, d_ref, o_ref):
        deg = jnp.maximum(jnp.sum(d_ref[...], axis=1), 1.0)
        o_ref[...] = (p_ref[0] + p_ref[1]) / deg[:, None]

    return pl.pallas_call(
        body,
        grid=(M // _RB,),
        in_specs=[pl.BlockSpec((2, _RB, H), lambda i: (0, i, 0)),
                  pl.BlockSpec((_RB, _NW), lambda i: (i, 0))],
        out_specs=_rb_spec(),
        out_shape=jax.ShapeDtypeStruct((M, H), jnp.float32),
    )(p, degp)


def _tc_mid(h1, p, degnp, xe, W2n, b2n, Wq, bq, Wk, bk, att):
    def body(h1_ref, p_ref, d_ref, xe_ref, w2n, bb2n, wq, bbq, wk, bbk,
             att_ref, ok, osq, osk):
        deg = jnp.maximum(jnp.sum(d_ref[...], axis=1), 1.0)
        n_agg = (p_ref[0] + p_ref[1]) / deg[:, None]
        h = _lrelu(jnp.dot(h1_ref[...] + n_agg, w2n[...],
                           preferred_element_type=jnp.float32) + bb2n[...])
        kk = jnp.dot(xe_ref[...], wk[...],
                     preferred_element_type=jnp.float32) + bbk[...]
        ok[...] = kk
        attq = att_ref[:H].reshape(H, 1)
        attk = att_ref[H:].reshape(H, 1)
        wqv = jnp.dot(wq[...], attq, preferred_element_type=jnp.float32)
        cq = jnp.sum(bbq[...] * attq[:, 0])
        i = pl.program_id(0)
        osq[pl.ds(i, 1), :] = (jnp.dot(h, wqv, preferred_element_type=jnp.float32)[:, 0]
                               + cq)[None, :]
        osk[pl.ds(i, 1), :] = jnp.dot(kk, attk,
                                      preferred_element_type=jnp.float32)[:, 0][None, :]

    return pl.pallas_call(
        body,
        grid=(N // _RB,),
        in_specs=[_rb_spec(), pl.BlockSpec((2, _RB, H), lambda i: (0, i, 0)),
                  pl.BlockSpec((_RB, _NW), lambda i: (i, 0)), _rb_spec(),
                  _full((H, H)), _full((H,)), _full((H, H)), _full((H,)),
                  _full((H, H)), _full((H,)), _full((2 * H,))],
        out_specs=[_rb_spec(), pl.BlockSpec((N // _RB, _RB), lambda i: (0, 0)),
                   pl.BlockSpec((M // _RB, _RB), lambda i: (0, 0))],
        out_shape=[jax.ShapeDtypeStruct((M, H), jnp.float32),
                   jax.ShapeDtypeStruct((N // _RB, _RB), jnp.float32),
                   jax.ShapeDtypeStruct((M // _RB, _RB), jnp.float32)],
    )(h1, p, degnp, xe, W2n, b2n, Wq, bq, Wk, bk, att)


def _tc_bv(sq2, sk2):
    def body(sq_ref, sk_ref, o_ref):
        m = jnp.max(sq_ref[...]) + jnp.max(sk_ref[...])
        o_ref[...] = jnp.broadcast_to(_lrelu(m), (16,))

    return pl.pallas_call(
        body,
        out_shape=jax.ShapeDtypeStruct((16,), jnp.float32),
    )(sq2, sk2)


def _tc_rec(denp):
    def body(d_ref, o_ref):
        o_ref[...] = 1.0 / (jnp.sum(d_ref[...], axis=0) + 1e-16)

    return pl.pallas_call(
        body,
        out_shape=jax.ShapeDtypeStruct((N,), jnp.float32),
    )(denp)


def _tc_add2(p):
    def body(p_ref, o_ref):
        o_ref[...] = p_ref[0] + p_ref[1]

    return pl.pallas_call(
        body,
        grid=(N // _RB,),
        in_specs=[pl.BlockSpec((2, _RB, H), lambda i: (0, i, 0))],
        out_specs=_rb_spec(),
        out_shape=jax.ShapeDtypeStruct((N, H), jnp.float32),
    )(p)


def _tc_stats(a, b):
    def body(a_ref, b_ref, o_ref):
        i = pl.program_id(0)
        av = a_ref[...]
        bv = b_ref[...]
        val = jnp.stack([jnp.sum(av, 0), jnp.sum(av * av, 0),
                         jnp.sum(bv, 0), jnp.sum(bv * bv, 0)], 0)

        @pl.when(i == 0)
        def _():
            o_ref[...] = val

        @pl.when(i > 0)
        def _():
            o_ref[...] = o_ref[...] + val

    return pl.pallas_call(
        body,
        grid=(M // _RB,),
        in_specs=[_rb_spec(), _rb_spec()],
        out_specs=pl.BlockSpec((4, H), lambda i: (0, 0)),
        out_shape=jax.ShapeDtypeStruct((4, H), jnp.float32),
    )(a, b)


def _gn_factors(s1, s2, gamma, beta, alpha, eps=1e-5):
    mean = s1 / M
    var = s2 / M - (2.0 * alpha - alpha * alpha) * mean * mean
    scale = gamma / jnp.sqrt(var + eps)
    return scale, beta - scale * alpha * mean


def _tc_fuse(hm, xe, st, g1, be1, a1, Wf, bf):
    def body(hm_ref, xe_ref, st_ref, g_ref, be_ref, a_ref, wf, bbf, ou, ost):
        i = pl.program_id(0)
        sc_a, off_a = _gn_factors(st_ref[0], st_ref[1], g_ref[0, :H],
                                  be_ref[0, :H], a_ref[0, :H])
        sc_b, off_b = _gn_factors(st_ref[2], st_ref[3], g_ref[0, H:],
                                  be_ref[0, H:], a_ref[0, H:])
        za = hm_ref[...] * sc_a + off_a
        zb = xe_ref[...] * sc_b + off_b
        u = _lrelu(jnp.dot(za, wf[:H], preferred_element_type=jnp.float32)
                   + jnp.dot(zb, wf[H:], preferred_element_type=jnp.float32)
                   + bbf[...])
        ou[...] = u
        val = jnp.stack([jnp.sum(u, 0), jnp.sum(u * u, 0)], 0)

        @pl.when(i == 0)
        def _():
            ost[...] = val

        @pl.when(i > 0)
        def _():
            ost[...] = ost[...] + val

    return pl.pallas_call(
        body,
        grid=(M // _RB,),
        in_specs=[_rb_spec(), _rb_spec(), _full((4, H)), _full((1, 2 * H)),
                  _full((1, 2 * H)), _full((1, 2 * H)), _full((2 * H, H)),
                  _full((H,))],
        out_specs=[_rb_spec(), pl.BlockSpec((2, H), lambda i: (0, 0))],
        out_shape=[jax.ShapeDtypeStruct((M, H), jnp.float32),
                   jax.ShapeDtypeStruct((2, H), jnp.float32)],
    )(hm, xe, st, g1.reshape(1, -1), be1.reshape(1, -1), a1.reshape(1, -1),
      Wf, bf)


def _tc_out(u, st, g2, be2, a2, Wc1, bc1, Wc2, bc2):
    def body(u_ref, st_ref, g_ref, be_ref, a_ref, wc1, bbc1, wc2, bbc2, o_ref):
        sc, off = _gn_factors(st_ref[0], st_ref[1], g_ref[0], be_ref[0],
                              a_ref[0])
        z = _lrelu(u_ref[...] * sc + off)
        z = _lrelu(jnp.dot(z, wc1[...], preferred_element_type=jnp.float32)
                   + bbc1[...])
        o_ref[...] = jnp.dot(z, wc2[...],
                             preferred_element_type=jnp.float32) + bbc2[...]

    D_OUT = Wc2.shape[1]
    return pl.pallas_call(
        body,
        grid=(M // _RB,),
        in_specs=[_rb_spec(), _full((2, H)), _full((1, H)), _full((1, H)),
                  _full((1, H)), _full((H, H)), _full((H,)),
                  _full((H, D_OUT)), _full((D_OUT,))],
        out_specs=pl.BlockSpec((_RB, D_OUT), lambda i: (i, 0)),
        out_shape=jax.ShapeDtypeStruct((M, D_OUT), jnp.float32),
    )(u, st, g2.reshape(1, -1), be2.reshape(1, -1), a2.reshape(1, -1),
      Wc1, bc1, Wc2, bc2)


def _seg_sum_body(table, gidx, sidx, zeros, out, gidx_v, sidx_v, rows0, rows1,
                  acc, gsem0, gsem1):
    c = lax.axis_index("c")
    s = lax.axis_index("s")
    w = s * _NC + c

    rows_per_sub = 624
    pltpu.sync_copy(zeros.at[pl.ds(s * rows_per_sub, rows_per_sub)],
                    acc.at[pl.ds(s * rows_per_sub, rows_per_sub)])

    @pl.when(s == 0)
    def _():
        pltpu.sync_copy(zeros.at[pl.ds(_NS * rows_per_sub, M - _NS * rows_per_sub)],
                        acc.at[pl.ds(_NS * rows_per_sub, M - _NS * rows_per_sub)])
    plsc.subcore_barrier()

    rows = (rows0, rows1)
    sems = (gsem0, gsem1)
    descs = [None, None]

    def start(j, b):
        descs[b] = pltpu.async_copy(table.at[gidx_v.at[j]], rows[b], sems[b])

    for blk in range(_ROWS_PER_W // _IDX_BLOCK):
        base = w * _ROWS_PER_W + blk * _IDX_BLOCK
        pltpu.sync_copy(gidx.at[pl.ds(base, _IDX_BLOCK)], gidx_v)
        pltpu.sync_copy(sidx.at[pl.ds(base, _IDX_BLOCK)], sidx_v)
        start(0, 0)
        for j in range(_IDX_BLOCK):
            b = j & 1
            if j + 1 < _IDX_BLOCK:
                start(j + 1, (j + 1) & 1)
            descs[b].wait()
            pltpu.sync_copy(rows[b], acc.at[sidx_v.at[j]], add=True)

    plsc.subcore_barrier()
    pltpu.sync_copy(acc.at[pl.ds(s * rows_per_sub, rows_per_sub)],
                    out.at[c].at[pl.ds(s * rows_per_sub, rows_per_sub)])

    @pl.when(s == 0)
    def _():
        pltpu.sync_copy(acc.at[pl.ds(_NS * rows_per_sub, M - _NS * rows_per_sub)],
                        out.at[c].at[pl.ds(_NS * rows_per_sub, M - _NS * rows_per_sub)])


_EPW = E // _NW
_NV = _EPW // 16
_SLOTS = 8


def _attn_ex_body(sq, sk, srcf, dstf, bv, ex_out, denp_out,
                  sq_v, sk_v, src_v, dst_v, ex_v, b_v, dacc):
    c = lax.axis_index("c")
    s = lax.axis_index("s")
    w = s * _NC + c
    base = w * _EPW
    pltpu.sync_copy(sq, sq_v)
    pltpu.sync_copy(sk, sk_v)
    pltpu.sync_copy(srcf.at[pl.ds(base, _EPW)], src_v)
    pltpu.sync_copy(dstf.at[pl.ds(base, _EPW)], dst_v)
    pltpu.sync_copy(bv, b_v)
    b16 = b_v[...]
    iota = lax.iota(jnp.int32, 16)
    mlo = iota < _SLOTS
    mhi = jnp.logical_not(mlo)
    slot_off = (iota % _SLOTS) * N

    def zbody(v, _):
        dacc[pl.ds(v * 16, 16)] = jnp.zeros((16,), jnp.float32)
        return 0

    lax.fori_loop(0, _SLOTS * N // 16, zbody, 0)

    def body(v, _):
        s16 = src_v[pl.ds(v * 16, 16)]
        d16 = dst_v[pl.ds(v * 16, 16)]
        a = plsc.load_gather(sq_v, [s16])
        b = plsc.load_gather(sk_v, [d16])
        sc = a + b
        sc = jnp.where(sc >= 0, sc, 0.01 * sc)
        e = jnp.exp(sc - b16)
        ex_v[pl.ds(v * 16, 16)] = e
        didx = slot_off + s16
        plsc.addupdate_scatter(dacc, [didx], e, mask=mlo)
        plsc.addupdate_scatter(dacc, [didx], e, mask=mhi)
        return 0

    lax.fori_loop(0, _NV, body, 0)

    def rbody(v, _):
        acc = dacc[pl.ds(v * 16, 16)]
        for k in range(1, _SLOTS):
            acc = acc + dacc[pl.ds(k * N + v * 16, 16)]
        dacc[pl.ds(v * 16, 16)] = acc
        return 0

    lax.fori_loop(0, N // 16, rbody, 0)
    pltpu.sync_copy(ex_v, ex_out.at[pl.ds(base, _EPW)])
    pltpu.sync_copy(dacc.at[pl.ds(0, N)], denp_out.at[pl.ds(w * N, N)])


def _attn_ex(sq, sk, srcf, dstf, bv):
    f = pl.kernel(
        _attn_ex_body,
        compiler_params=pltpu.CompilerParams(needs_layout_passes=False),
        out_type=(jax.ShapeDtypeStruct((E,), jnp.float32),
                  jax.ShapeDtypeStruct((_NW * N,), jnp.float32)),
        mesh=plsc.VectorSubcoreMesh(core_axis_name="c", subcore_axis_name="s"),
        scratch_types=[
            pltpu.VMEM((N,), jnp.float32),
            pltpu.VMEM((M,), jnp.float32),
            pltpu.VMEM((_EPW,), jnp.int32),
            pltpu.VMEM((_EPW,), jnp.int32),
            pltpu.VMEM((_EPW,), jnp.float32),
            pltpu.VMEM((16,), jnp.float32),
            pltpu.VMEM((_SLOTS * N,), jnp.float32),
        ],
    )
    return f(sq, sk, srcf, dstf, bv)


def _deg_body(idxf, out, idx_v, dacc):
    c = lax.axis_index("c")
    s = lax.axis_index("s")
    w = s * _NC + c
    base = w * _EPW
    pltpu.sync_copy(idxf.at[pl.ds(base, _EPW)], idx_v)
    iota = lax.iota(jnp.int32, 16)
    mlo = iota < _SLOTS
    mhi = jnp.logical_not(mlo)
    slot_off = (iota % _SLOTS) * N
    ones16 = jnp.ones((16,), jnp.float32)

    def zbody(v, _):
        dacc[pl.ds(v * 16, 16)] = jnp.zeros((16,), jnp.float32)
        return 0

    lax.fori_loop(0, _SLOTS * N // 16, zbody, 0)

    def body(v, _):
        i16 = idx_v[pl.ds(v * 16, 16)]
        didx = slot_off + i16
        plsc.addupdate_scatter(dacc, [didx], ones16, mask=mlo)
        plsc.addupdate_scatter(dacc, [didx], ones16, mask=mhi)
        return 0

    lax.fori_loop(0, _NV, body, 0)

    def rbody(v, _):
        acc = dacc[pl.ds(v * 16, 16)]
        for k in range(1, _SLOTS):
            acc = acc + dacc[pl.ds(k * N + v * 16, 16)]
        dacc[pl.ds(v * 16, 16)] = acc
        return 0

    lax.fori_loop(0, N // 16, rbody, 0)
    pltpu.sync_copy(dacc.at[pl.ds(0, N)], out.at[pl.ds(w * N, N)])


def _deg(idxf):
    f = pl.kernel(
        _deg_body,
        compiler_params=pltpu.CompilerParams(needs_layout_passes=False),
        out_type=jax.ShapeDtypeStruct((_NW * N,), jnp.float32),
        mesh=plsc.VectorSubcoreMesh(core_axis_name="c", subcore_axis_name="s"),
        scratch_types=[
            pltpu.VMEM((_EPW,), jnp.int32),
            pltpu.VMEM((_SLOTS * N,), jnp.float32),
        ],
    )
    return f(idxf)


_HN_BLK = 16


def _hn_body(ktab, src2d, dst2d, ex2d, rec, zeros, out,
             src_v, dst_v, ex_v, rec_v, al_v, rows0, rows1, acc, gsem0, gsem1):
    c = lax.axis_index("c")
    s = lax.axis_index("s")
    w = s * _NC + c

    rows_per_sub = 624
    pltpu.sync_copy(zeros.at[pl.ds(s * rows_per_sub, rows_per_sub)],
                    acc.at[pl.ds(s * rows_per_sub, rows_per_sub)])

    @pl.when(s == 0)
    def _():
        pltpu.sync_copy(zeros.at[pl.ds(_NS * rows_per_sub, N - _NS * rows_per_sub)],
                        acc.at[pl.ds(_NS * rows_per_sub, N - _NS * rows_per_sub)])

    pltpu.sync_copy(rec, rec_v)
    plsc.subcore_barrier()

    rows = (rows0, rows1)
    sems = (gsem0, gsem1)
    descs = [None, None]

    def start(j, b):
        descs[b] = pltpu.async_copy(ktab.at[dst_v.at[j]], rows[b], sems[b])

    offs = [v * 16 for v in range(7)] + [_CHUNK - 16]
    for blk in range(_ROWS_PER_W // _HN_BLK):
        rbase = w * _ROWS_PER_W + blk * _HN_BLK
        pltpu.sync_copy(src2d.at[pl.ds(rbase, _HN_BLK)], src_v)
        pltpu.sync_copy(dst2d.at[pl.ds(rbase, _HN_BLK)], dst_v)
        pltpu.sync_copy(ex2d.at[pl.ds(rbase, _HN_BLK)], ex_v)
        for j in range(_HN_BLK):
            for off in offs:
                s16 = src_v[j, pl.ds(off, 16)]
                e16 = ex_v[j, pl.ds(off, 16)]
                al_v[pl.ds(j * _CHUNK + off, 16)] = \
                    e16 * plsc.load_gather(rec_v, [s16])
        start(0, 0)
        for j in range(_HN_BLK):
            b = j & 1
            if j + 1 < _HN_BLK:
                start(j + 1, (j + 1) & 1)
            descs[b].wait()

            def ebody(i, _):
                sp = plsc.load_gather(al_v, [jnp.full((16,), j * _CHUNK, jnp.int32) + i])
                for t in range(8):
                    rows[b][i, pl.ds(t * 16, 16)] = rows[b][i, pl.ds(t * 16, 16)] * sp
                return 0

            lax.fori_loop(0, _CHUNK, ebody, 0)
            pltpu.sync_copy(rows[b], acc.at[src_v.at[j]], add=True)

    plsc.subcore_barrier()
    pltpu.sync_copy(acc.at[pl.ds(s * rows_per_sub, rows_per_sub)],
                    out.at[c].at[pl.ds(s * rows_per_sub, rows_per_sub)])

    @pl.when(s == 0)
    def _():
        pltpu.sync_copy(acc.at[pl.ds(_NS * rows_per_sub, N - _NS * rows_per_sub)],
                        out.at[c].at[pl.ds(_NS * rows_per_sub, N - _NS * rows_per_sub)])


def _hn_pass(ktab, src2d, dst2d, ex2d, rec, zeros):
    f = pl.kernel(
        _hn_body,
        compiler_params=pltpu.CompilerParams(needs_layout_passes=False),
        out_type=jax.ShapeDtypeStruct((_NC, N, H), jnp.float32),
        mesh=plsc.VectorSubcoreMesh(core_axis_name="c", subcore_axis_name="s"),
        scratch_types=[
            pltpu.VMEM((_HN_BLK, _CHUNK), jnp.int32),
            pltpu.VMEM((_HN_BLK, _CHUNK), jnp.int32),
            pltpu.VMEM((_HN_BLK, _CHUNK), jnp.float32),
            pltpu.VMEM((N,), jnp.float32),
            pltpu.VMEM((_HN_BLK * _CHUNK,), jnp.float32),
            pltpu.VMEM((_CHUNK, H), jnp.float32),
            pltpu.VMEM((_CHUNK, H), jnp.float32),
            pltpu.VMEM_SHARED((N, H), jnp.float32),
            pltpu.SemaphoreType.DMA,
            pltpu.SemaphoreType.DMA,
        ],
    )
    return f(ktab, src2d, dst2d, ex2d, rec, zeros)


_MIN_OWN = 312
_MIN_ACC = 328
_SC_BLK = 4000
_MCAP = 8192


def _min_body(hn, srcf, dstf, out, sb0, db0, sb1, db1, msrc, mdlo,
              rows0, rows1, acc, ssem0, ssem1, gsem0, gsem1):
    c = lax.axis_index("c")
    s = lax.axis_index("s")
    w = s * _NC + c
    lo = w * _MIN_OWN
    hi = jnp.where(w == _NW - 1, M, lo + _MIN_OWN)
    lo16 = jnp.broadcast_to(lo, (16,))
    hi16 = jnp.broadcast_to(hi, (16,))
    sbufs = (sb0, sb1)
    dbufs = (db0, db1)
    ssems = (ssem0, ssem1)
    rbufs = (rows0, rows1)
    gsems = (gsem0, gsem1)

    def zb(v, _):
        msrc[pl.ds(v * 16, 16)] = jnp.zeros((16,), jnp.int32)
        return 0

    lax.fori_loop(0, _MCAP // 16, zb, 0)

    inf16 = jnp.full((16,), jnp.inf, jnp.float32)

    def accinit(v, _):
        acc[v // 8, pl.ds((v % 8) * 16, 16)] = inf16
        return 0

    lax.fori_loop(0, _MIN_ACC * 8, accinit, 0)

    nblk = E // _SC_BLK
    iota16 = lax.iota(jnp.int32, 16)

    def stage_start(blk, b):
        pltpu.async_copy(srcf.at[pl.ds(blk * _SC_BLK, _SC_BLK)], sbufs[b], ssems[b])
        pltpu.async_copy(dstf.at[pl.ds(blk * _SC_BLK, _SC_BLK)], dbufs[b], ssems[b])

    def stage_wait(b):
        pltpu.make_async_copy(srcf.at[pl.ds(0, _SC_BLK)], sbufs[b], ssems[b]).wait()
        pltpu.make_async_copy(dstf.at[pl.ds(0, _SC_BLK)], dbufs[b], ssems[b]).wait()

    def gather_start(cidx, rb):
        pltpu.async_copy(hn.at[msrc.at[pl.ds(cidx * 128, 128)]],
                         rbufs[rb], gsems[rb])

    def gather_wait(rb):
        pltpu.make_async_copy(hn.at[msrc.at[pl.ds(0, 128)]],
                              rbufs[rb], gsems[rb]).wait()

    def rmw_edge(cidx, i, rref):
        dl = plsc.load_gather(mdlo, [jnp.full((16,), 0, jnp.int32)
                                     + (cidx * 128 + i)])
        for t in range(8):
            col = iota16 + (t * 16)
            cur = plsc.load_gather(acc, [dl, col])
            r = rref[i, pl.ds(t * 16, 16)]
            plsc.store_scatter(acc, [dl, col], jnp.minimum(cur, r))

    def rmw_full(cidx, rb):
        def e4(i, _):
            for k in range(4):
                rmw_edge(cidx, 4 * i + k, rbufs[rb])
            return 0

        lax.fori_loop(0, 32, e4, 0)

    def process_full(cnt):
        nch = cnt // 128
        gather_start(0, 0)

        def c2(p2, _):
            c0 = 2 * p2
            c1 = c0 + 1
            gather_start(c1, 1)
            gather_wait(0)
            rmw_full(c0, 0)
            gather_start(c0 + 2, 0)

            gather_wait(1)

            @pl.when(c1 < nch)
            def _():
                rmw_full(c1, 1)

            return 0

        lax.fori_loop(0, (nch + 1) // 2, c2, 0)
        gather_wait(0)
        for k in range(8):
            msrc[pl.ds(k * 16, 16)] = msrc[pl.ds(nch * 128 + k * 16, 16)]
            mdlo[pl.ds(k * 16, 16)] = mdlo[pl.ds(nch * 128 + k * 16, 16)]
        return cnt - nch * 128

    def scan(b, cnt0):
        def scan_body(v, cnt):
            s16 = sbufs[b][pl.ds(v * 16, 16)]
            d16 = dbufs[b][pl.ds(v * 16, 16)]
            m = jnp.logical_and(d16 >= lo16, d16 < hi16)
            plsc.store_compressed(msrc.at[pl.ds(cnt, 16)], s16, mask=m)
            plsc.store_compressed(mdlo.at[pl.ds(cnt, 16)], d16 - lo16, mask=m)
            return cnt + jnp.sum(m.astype(jnp.int32))

        return lax.fori_loop(0, _SC_BLK // 16, scan_body, cnt0)

    stage_start(0, 0)
    _FLUSH = _MCAP - _SC_BLK

    def maybe_flush(cnt):
        return lax.cond(cnt >= _FLUSH, process_full, lambda c: c, cnt)

    def blk2(p, cnt):
        b0 = 2 * p
        stage_start(b0 + 1, 1)
        stage_wait(0)
        cnt = maybe_flush(scan(0, cnt))

        @pl.when(b0 + 2 < nblk)
        def _():
            stage_start(b0 + 2, 0)

        stage_wait(1)
        cnt = maybe_flush(scan(1, cnt))
        return cnt

    cnt = lax.fori_loop(0, nblk // 2, blk2, 0)

    def fchunk(cidx, cnt):
        pltpu.async_copy(hn.at[msrc.at[pl.ds(cidx * 128, 128)]],
                         rows0, gsem0).wait()
        nb = jnp.minimum(cnt - cidx * 128, 128)

        def e1(i, _):
            rmw_edge(cidx, i, rows0)
            return 0

        lax.fori_loop(0, nb, e1, 0)
        return cnt

    lax.fori_loop(0, (cnt + 127) // 128, fchunk, cnt)

    def fin(v, _):
        r = v // 8
        off = (v % 8) * 16
        val = acc[r, pl.ds(off, 16)]
        acc[r, pl.ds(off, 16)] = jnp.where(val * 0.0 == 0.0, val, 0.0)
        return 0

    lax.fori_loop(0, _MIN_ACC * 8, fin, 0)
    pltpu.sync_copy(acc.at[pl.ds(0, _MIN_OWN)], out.at[pl.ds(lo, _MIN_OWN)])

    @pl.when(w == _NW - 1)
    def _():
        pltpu.sync_copy(acc.at[pl.ds(_MIN_OWN, _MIN_ACC - _MIN_OWN)],
                        out.at[pl.ds(M - (_MIN_ACC - _MIN_OWN), _MIN_ACC - _MIN_OWN)])


def _min_pass(hn, srcf, dstf):
    f = pl.kernel(
        _min_body,
        compiler_params=pltpu.CompilerParams(needs_layout_passes=False),
        out_type=jax.ShapeDtypeStruct((M, H), jnp.float32),
        mesh=plsc.VectorSubcoreMesh(core_axis_name="c", subcore_axis_name="s"),
        scratch_types=[
            pltpu.VMEM((_SC_BLK,), jnp.int32),
            pltpu.VMEM((_SC_BLK,), jnp.int32),
            pltpu.VMEM((_SC_BLK,), jnp.int32),
            pltpu.VMEM((_SC_BLK,), jnp.int32),
            pltpu.VMEM((_MCAP,), jnp.int32),
            pltpu.VMEM((_MCAP,), jnp.int32),
            pltpu.VMEM((128, H), jnp.float32),
            pltpu.VMEM((128, H), jnp.float32),
            pltpu.VMEM((_MIN_ACC, H), jnp.float32),
            pltpu.SemaphoreType.DMA,
            pltpu.SemaphoreType.DMA,
            pltpu.SemaphoreType.DMA,
            pltpu.SemaphoreType.DMA,
        ],
    )
    return f(hn, srcf, dstf)


@functools.partial(jax.jit, static_argnums=())
def _seg_sum(table, gidx_rows, sidx_rows, zeros):
    f = pl.kernel(
        _seg_sum_body,
        out_type=jax.ShapeDtypeStruct((_NC, M, H), jnp.float32),
        mesh=plsc.VectorSubcoreMesh(core_axis_name="c", subcore_axis_name="s"),
        scratch_types=[
            pltpu.VMEM((_IDX_BLOCK, _CHUNK), jnp.int32),
            pltpu.VMEM((_IDX_BLOCK, _CHUNK), jnp.int32),
            pltpu.VMEM((_CHUNK, H), jnp.float32),
            pltpu.VMEM((_CHUNK, H), jnp.float32),
            pltpu.VMEM_SHARED((M, H), jnp.float32),
            pltpu.SemaphoreType.DMA,
            pltpu.SemaphoreType.DMA,
        ],
    )
    return f(table, gidx_rows, sidx_rows, zeros)


def kernel(x, x_struct, x_e, edge_index, W1e, b1e, W2e, b2e, W1n, b1n, W2n, b2n, Wq, bq, Wk, bk, att, g1, be1, a1, Wf, bf, g2, be2, a2, Wc1, bc1, Wc2, bc2):
    src = edge_index[0]
    dst = edge_index[1]
    src_rows = src.reshape(E // _CHUNK, _CHUNK)
    dst_rows = dst.reshape(E // _CHUNK, _CHUNK)
    zeros_mh = jnp.zeros((M, H), jnp.float32)

    xe, h1 = _tc_pre(x_e, x, W1e, b1e, W2e, b2e, W1n, b1n)
    degep = _deg(dst).reshape(_NW, M).T
    degnp = _deg(src).reshape(_NW, N).T
    p = _seg_sum(h1, src_rows, dst_rows, zeros_mh)
    e_agg = _tc_agg_div(p, degep)
    p = _seg_sum(e_agg, dst_rows, src_rows, zeros_mh)
    k, sq, sk = _tc_mid(h1, p, degnp, xe, W2n, b2n, Wq, bq, Wk, bk, att)
    sq = sq.reshape(N)
    sk = sk.reshape(M)
    bv = _tc_bv(sq.reshape(8, N // 8), sk.reshape(8, M // 8))
    ex, denp = _attn_ex(sq, sk, src, dst, bv)
    rec = _tc_rec(denp.reshape(_NW, N))
    ex_rows = ex.reshape(E // _CHUNK, _CHUNK)
    p = _hn_pass(k, src_rows, dst_rows, ex_rows, rec, zeros_mh)
    hn = _tc_add2(p)
    hm = _min_pass(hn, src, dst)
    st1 = _tc_stats(hm, xe)
    u, st2 = _tc_fuse(hm, xe, st1, g1, be1, a1, Wf, bf)
    return _tc_out(u, st2, g2, be2, a2, Wc1, bc1, Wc2, bc2)


# speedup vs baseline: 3.3145x; 1.0465x over previous
import functools

import jax
import jax.numpy as jnp
from jax import lax
from jax.experimental import pallas as pl
from jax.experimental.pallas import tpu as pltpu
from jax.experimental.pallas import tpu_sc as plsc

N = 10000
M = 10000
E = 320000
H = 128

_NC = 2
_NS = 16
_NW = _NC * _NS
_CHUNK = 125
_ROWS_PER_W = E // _CHUNK // _NW
_IDX_BLOCK = 40


def _lrelu(v):
    return jnp.where(v >= 0, v, 0.01 * v)


_RB = 2000


def _rb_spec():
    return pl.BlockSpec((_RB, H), lambda i: (i, 0))


def _full(shape):
    nd = len(shape)
    return pl.BlockSpec(shape, lambda i: (0,) * nd)


def _tc_pre(x_e, x, W1e, b1e, W2e, b2e, W1n, b1n):
    def body(xe_ref, x_ref, w1e, bb1e, w2e, bb2e, w1n, bb1n, oxe, oh1):
        t = _lrelu(jnp.dot(xe_ref[...], w1e[...],
                           preferred_element_type=jnp.float32) + bb1e[...])
        oxe[...] = _lrelu(jnp.dot(t, w2e[...],
                                  preferred_element_type=jnp.float32) + bb2e[...])
        oh1[...] = _lrelu(jnp.dot(x_ref[...], w1n[...],
                                  preferred_element_type=jnp.float32) + bb1n[...])

    return pl.pallas_call(
        body,
        grid=(M // _RB,),
        in_specs=[_rb_spec(), _rb_spec(), _full((H, H)), _full((H,)),
                  _full((H, H)), _full((H,)), _full((H, H)), _full((H,))],
        out_specs=[_rb_spec(), _rb_spec()],
        out_shape=[jax.ShapeDtypeStruct((M, H), jnp.float32),
                   jax.ShapeDtypeStruct((N, H), jnp.float32)],
    )(x_e, x, W1e, b1e, W2e, b2e, W1n, b1n)


def _tc_agg_div(p, degp):
    def body(p_ref, d_ref, o_ref):
        deg = jnp.maximum(jnp.sum(d_ref[...], axis=1), 1.0)
        o_ref[...] = (p_ref[0] + p_ref[1]) / deg[:, None]

    return pl.pallas_call(
        body,
        grid=(M // _RB,),
        in_specs=[pl.BlockSpec((2, _RB, H), lambda i: (0, i, 0)),
                  pl.BlockSpec((_RB, _NW), lambda i: (i, 0))],
        out_specs=_rb_spec(),
        out_shape=jax.ShapeDtypeStruct((M, H), jnp.float32),
    )(p, degp)


def _tc_mid(h1, p, degnp, xe, W2n, b2n, Wq, bq, Wk, bk, att):
    def body(h1_ref, p_ref, d_ref, xe_ref, w2n, bb2n, wq, bbq, wk, bbk,
             att_ref, ok, osq, osk):
        deg = jnp.maximum(jnp.sum(d_ref[...], axis=1), 1.0)
        n_agg = (p_ref[0] + p_ref[1]) / deg[:, None]
        h = _lrelu(jnp.dot(h1_ref[...] + n_agg, w2n[...],
                           preferred_element_type=jnp.float32) + bb2n[...])
        kk = jnp.dot(xe_ref[...], wk[...],
                     preferred_element_type=jnp.float32) + bbk[...]
        ok[...] = kk
        attq = att_ref[:H].reshape(H, 1)
        attk = att_ref[H:].reshape(H, 1)
        wqv = jnp.dot(wq[...], attq, preferred_element_type=jnp.float32)
        cq = jnp.sum(bbq[...] * attq[:, 0])
        i = pl.program_id(0)
        osq[pl.ds(i, 1), :] = (jnp.dot(h, wqv, preferred_element_type=jnp.float32)[:, 0]
                               + cq)[None, :]
        osk[pl.ds(i, 1), :] = jnp.dot(kk, attk,
                                      preferred_element_type=jnp.float32)[:, 0][None, :]

    return pl.pallas_call(
        body,
        grid=(N // _RB,),
        in_specs=[_rb_spec(), pl.BlockSpec((2, _RB, H), lambda i: (0, i, 0)),
                  pl.BlockSpec((_RB, _NW), lambda i: (i, 0)), _rb_spec(),
                  _full((H, H)), _full((H,)), _full((H, H)), _full((H,)),
                  _full((H, H)), _full((H,)), _full((2 * H,))],
        out_specs=[_rb_spec(), pl.BlockSpec((N // _RB, _RB), lambda i: (0, 0)),
                   pl.BlockSpec((M // _RB, _RB), lambda i: (0, 0))],
        out_shape=[jax.ShapeDtypeStruct((M, H), jnp.float32),
                   jax.ShapeDtypeStruct((N // _RB, _RB), jnp.float32),
                   jax.ShapeDtypeStruct((M // _RB, _RB), jnp.float32)],
    )(h1, p, degnp, xe, W2n, b2n, Wq, bq, Wk, bk, att)


def _tc_bv(sq2, sk2):
    def body(sq_ref, sk_ref, o_ref):
        m = jnp.max(sq_ref[...]) + jnp.max(sk_ref[...])
        o_ref[...] = jnp.broadcast_to(_lrelu(m), (16,))

    return pl.pallas_call(
        body,
        out_shape=jax.ShapeDtypeStruct((16,), jnp.float32),
    )(sq2, sk2)


def _tc_rec(denp):
    def body(d_ref, o_ref):
        o_ref[...] = 1.0 / (jnp.sum(d_ref[...], axis=0) + 1e-16)

    return pl.pallas_call(
        body,
        out_shape=jax.ShapeDtypeStruct((N,), jnp.float32),
    )(denp)


def _tc_add2(p):
    def body(p_ref, o_ref):
        o_ref[...] = p_ref[0] + p_ref[1]

    return pl.pallas_call(
        body,
        grid=(N // _RB,),
        in_specs=[pl.BlockSpec((2, _RB, H), lambda i: (0, i, 0))],
        out_specs=_rb_spec(),
        out_shape=jax.ShapeDtypeStruct((N, H), jnp.float32),
    )(p)


def _tc_stats(a, b):
    def body(a_ref, b_ref, o_ref):
        i = pl.program_id(0)
        av = a_ref[...]
        bv = b_ref[...]
        val = jnp.stack([jnp.sum(av, 0), jnp.sum(av * av, 0),
                         jnp.sum(bv, 0), jnp.sum(bv * bv, 0)], 0)

        @pl.when(i == 0)
        def _():
            o_ref[...] = val

        @pl.when(i > 0)
        def _():
            o_ref[...] = o_ref[...] + val

    return pl.pallas_call(
        body,
        grid=(M // _RB,),
        in_specs=[_rb_spec(), _rb_spec()],
        out_specs=pl.BlockSpec((4, H), lambda i: (0, 0)),
        out_shape=jax.ShapeDtypeStruct((4, H), jnp.float32),
    )(a, b)


def _gn_factors(s1, s2, gamma, beta, alpha, eps=1e-5):
    mean = s1 / M
    var = s2 / M - (2.0 * alpha - alpha * alpha) * mean * mean
    scale = gamma / jnp.sqrt(var + eps)
    return scale, beta - scale * alpha * mean


def _tc_fuse(hm, xe, st, g1, be1, a1, Wf, bf):
    def body(hm_ref, xe_ref, st_ref, g_ref, be_ref, a_ref, wf, bbf, ou, ost):
        i = pl.program_id(0)
        sc_a, off_a = _gn_factors(st_ref[0], st_ref[1], g_ref[0, :H],
                                  be_ref[0, :H], a_ref[0, :H])
        sc_b, off_b = _gn_factors(st_ref[2], st_ref[3], g_ref[0, H:],
                                  be_ref[0, H:], a_ref[0, H:])
        za = hm_ref[...] * sc_a + off_a
        zb = xe_ref[...] * sc_b + off_b
        u = _lrelu(jnp.dot(za, wf[:H], preferred_element_type=jnp.float32)
                   + jnp.dot(zb, wf[H:], preferred_element_type=jnp.float32)
                   + bbf[...])
        ou[...] = u
        val = jnp.stack([jnp.sum(u, 0), jnp.sum(u * u, 0)], 0)

        @pl.when(i == 0)
        def _():
            ost[...] = val

        @pl.when(i > 0)
        def _():
            ost[...] = ost[...] + val

    return pl.pallas_call(
        body,
        grid=(M // _RB,),
        in_specs=[_rb_spec(), _rb_spec(), _full((4, H)), _full((1, 2 * H)),
                  _full((1, 2 * H)), _full((1, 2 * H)), _full((2 * H, H)),
                  _full((H,))],
        out_specs=[_rb_spec(), pl.BlockSpec((2, H), lambda i: (0, 0))],
        out_shape=[jax.ShapeDtypeStruct((M, H), jnp.float32),
                   jax.ShapeDtypeStruct((2, H), jnp.float32)],
    )(hm, xe, st, g1.reshape(1, -1), be1.reshape(1, -1), a1.reshape(1, -1),
      Wf, bf)


def _tc_out(u, st, g2, be2, a2, Wc1, bc1, Wc2, bc2):
    def body(u_ref, st_ref, g_ref, be_ref, a_ref, wc1, bbc1, wc2, bbc2, o_ref):
        sc, off = _gn_factors(st_ref[0], st_ref[1], g_ref[0], be_ref[0],
                              a_ref[0])
        z = _lrelu(u_ref[...] * sc + off)
        z = _lrelu(jnp.dot(z, wc1[...], preferred_element_type=jnp.float32)
                   + bbc1[...])
        o_ref[...] = jnp.dot(z, wc2[...],
                             preferred_element_type=jnp.float32) + bbc2[...]

    D_OUT = Wc2.shape[1]
    return pl.pallas_call(
        body,
        grid=(M // _RB,),
        in_specs=[_rb_spec(), _full((2, H)), _full((1, H)), _full((1, H)),
                  _full((1, H)), _full((H, H)), _full((H,)),
                  _full((H, D_OUT)), _full((D_OUT,))],
        out_specs=pl.BlockSpec((_RB, D_OUT), lambda i: (i, 0)),
        out_shape=jax.ShapeDtypeStruct((M, D_OUT), jnp.float32),
    )(u, st, g2.reshape(1, -1), be2.reshape(1, -1), a2.reshape(1, -1),
      Wc1, bc1, Wc2, bc2)


def _seg_sum_body(table, gidx, sidx, zeros, out, gidx_v, sidx_v, rows0, rows1,
                  acc, gsem0, gsem1):
    c = lax.axis_index("c")
    s = lax.axis_index("s")
    w = s * _NC + c

    rows_per_sub = 624
    pltpu.sync_copy(zeros.at[pl.ds(s * rows_per_sub, rows_per_sub)],
                    acc.at[pl.ds(s * rows_per_sub, rows_per_sub)])

    @pl.when(s == 0)
    def _():
        pltpu.sync_copy(zeros.at[pl.ds(_NS * rows_per_sub, M - _NS * rows_per_sub)],
                        acc.at[pl.ds(_NS * rows_per_sub, M - _NS * rows_per_sub)])
    plsc.subcore_barrier()

    rows = (rows0, rows1)
    sems = (gsem0, gsem1)
    descs = [None, None]

    def start(j, b):
        descs[b] = pltpu.async_copy(table.at[gidx_v.at[j]], rows[b], sems[b])

    for blk in range(_ROWS_PER_W // _IDX_BLOCK):
        base = w * _ROWS_PER_W + blk * _IDX_BLOCK
        pltpu.sync_copy(gidx.at[pl.ds(base, _IDX_BLOCK)], gidx_v)
        pltpu.sync_copy(sidx.at[pl.ds(base, _IDX_BLOCK)], sidx_v)
        start(0, 0)
        for j in range(_IDX_BLOCK):
            b = j & 1
            if j + 1 < _IDX_BLOCK:
                start(j + 1, (j + 1) & 1)
            descs[b].wait()
            pltpu.sync_copy(rows[b], acc.at[sidx_v.at[j]], add=True)

    plsc.subcore_barrier()
    pltpu.sync_copy(acc.at[pl.ds(s * rows_per_sub, rows_per_sub)],
                    out.at[c].at[pl.ds(s * rows_per_sub, rows_per_sub)])

    @pl.when(s == 0)
    def _():
        pltpu.sync_copy(acc.at[pl.ds(_NS * rows_per_sub, M - _NS * rows_per_sub)],
                        out.at[c].at[pl.ds(_NS * rows_per_sub, M - _NS * rows_per_sub)])


_EPW = E // _NW
_NV = _EPW // 16
_SLOTS = 8


def _attn_ex_body(sq, sk, srcf, dstf, bv, ex_out, denp_out,
                  sq_v, sk_v, src_v, dst_v, ex_v, b_v, dacc):
    c = lax.axis_index("c")
    s = lax.axis_index("s")
    w = s * _NC + c
    base = w * _EPW
    pltpu.sync_copy(sq, sq_v)
    pltpu.sync_copy(sk, sk_v)
    pltpu.sync_copy(srcf.at[pl.ds(base, _EPW)], src_v)
    pltpu.sync_copy(dstf.at[pl.ds(base, _EPW)], dst_v)
    pltpu.sync_copy(bv, b_v)
    b16 = b_v[...]
    iota = lax.iota(jnp.int32, 16)
    mlo = iota < _SLOTS
    mhi = jnp.logical_not(mlo)
    slot_off = (iota % _SLOTS) * N

    def zbody(v, _):
        dacc[pl.ds(v * 16, 16)] = jnp.zeros((16,), jnp.float32)
        return 0

    lax.fori_loop(0, _SLOTS * N // 16, zbody, 0)

    def body(v, _):
        s16 = src_v[pl.ds(v * 16, 16)]
        d16 = dst_v[pl.ds(v * 16, 16)]
        a = plsc.load_gather(sq_v, [s16])
        b = plsc.load_gather(sk_v, [d16])
        sc = a + b
        sc = jnp.where(sc >= 0, sc, 0.01 * sc)
        e = jnp.exp(sc - b16)
        ex_v[pl.ds(v * 16, 16)] = e
        didx = slot_off + s16
        plsc.addupdate_scatter(dacc, [didx], e, mask=mlo)
        plsc.addupdate_scatter(dacc, [didx], e, mask=mhi)
        return 0

    lax.fori_loop(0, _NV, body, 0)

    def rbody(v, _):
        acc = dacc[pl.ds(v * 16, 16)]
        for k in range(1, _SLOTS):
            acc = acc + dacc[pl.ds(k * N + v * 16, 16)]
        dacc[pl.ds(v * 16, 16)] = acc
        return 0

    lax.fori_loop(0, N // 16, rbody, 0)
    pltpu.sync_copy(ex_v, ex_out.at[pl.ds(base, _EPW)])
    pltpu.sync_copy(dacc.at[pl.ds(0, N)], denp_out.at[pl.ds(w * N, N)])


def _attn_ex(sq, sk, srcf, dstf, bv):
    f = pl.kernel(
        _attn_ex_body,
        compiler_params=pltpu.CompilerParams(needs_layout_passes=False),
        out_type=(jax.ShapeDtypeStruct((E,), jnp.float32),
                  jax.ShapeDtypeStruct((_NW * N,), jnp.float32)),
        mesh=plsc.VectorSubcoreMesh(core_axis_name="c", subcore_axis_name="s"),
        scratch_types=[
            pltpu.VMEM((N,), jnp.float32),
            pltpu.VMEM((M,), jnp.float32),
            pltpu.VMEM((_EPW,), jnp.int32),
            pltpu.VMEM((_EPW,), jnp.int32),
            pltpu.VMEM((_EPW,), jnp.float32),
            pltpu.VMEM((16,), jnp.float32),
            pltpu.VMEM((_SLOTS * N,), jnp.float32),
        ],
    )
    return f(sq, sk, srcf, dstf, bv)


def _deg_body(idxf, out, idx_v, dacc):
    c = lax.axis_index("c")
    s = lax.axis_index("s")
    w = s * _NC + c
    base = w * _EPW
    pltpu.sync_copy(idxf.at[pl.ds(base, _EPW)], idx_v)
    iota = lax.iota(jnp.int32, 16)
    mlo = iota < _SLOTS
    mhi = jnp.logical_not(mlo)
    slot_off = (iota % _SLOTS) * N
    ones16 = jnp.ones((16,), jnp.float32)

    def zbody(v, _):
        dacc[pl.ds(v * 16, 16)] = jnp.zeros((16,), jnp.float32)
        return 0

    lax.fori_loop(0, _SLOTS * N // 16, zbody, 0)

    def body(v, _):
        i16 = idx_v[pl.ds(v * 16, 16)]
        didx = slot_off + i16
        plsc.addupdate_scatter(dacc, [didx], ones16, mask=mlo)
        plsc.addupdate_scatter(dacc, [didx], ones16, mask=mhi)
        return 0

    lax.fori_loop(0, _NV, body, 0)

    def rbody(v, _):
        acc = dacc[pl.ds(v * 16, 16)]
        for k in range(1, _SLOTS):
            acc = acc + dacc[pl.ds(k * N + v * 16, 16)]
        dacc[pl.ds(v * 16, 16)] = acc
        return 0

    lax.fori_loop(0, N // 16, rbody, 0)
    pltpu.sync_copy(dacc.at[pl.ds(0, N)], out.at[pl.ds(w * N, N)])


def _deg(idxf):
    f = pl.kernel(
        _deg_body,
        compiler_params=pltpu.CompilerParams(needs_layout_passes=False),
        out_type=jax.ShapeDtypeStruct((_NW * N,), jnp.float32),
        mesh=plsc.VectorSubcoreMesh(core_axis_name="c", subcore_axis_name="s"),
        scratch_types=[
            pltpu.VMEM((_EPW,), jnp.int32),
            pltpu.VMEM((_SLOTS * N,), jnp.float32),
        ],
    )
    return f(idxf)


_HN_BLK = 16


def _hn_body(ktab, src2d, dst2d, ex2d, rec, zeros, out,
             src_v, dst_v, ex_v, rec_v, al_v, rows0, rows1, acc, gsem0, gsem1):
    c = lax.axis_index("c")
    s = lax.axis_index("s")
    w = s * _NC + c

    rows_per_sub = 624
    pltpu.sync_copy(zeros.at[pl.ds(s * rows_per_sub, rows_per_sub)],
                    acc.at[pl.ds(s * rows_per_sub, rows_per_sub)])

    @pl.when(s == 0)
    def _():
        pltpu.sync_copy(zeros.at[pl.ds(_NS * rows_per_sub, N - _NS * rows_per_sub)],
                        acc.at[pl.ds(_NS * rows_per_sub, N - _NS * rows_per_sub)])

    pltpu.sync_copy(rec, rec_v)
    plsc.subcore_barrier()

    rows = (rows0, rows1)
    sems = (gsem0, gsem1)
    descs = [None, None]

    def start(j, b):
        descs[b] = pltpu.async_copy(ktab.at[dst_v.at[j]], rows[b], sems[b])

    offs = [v * 16 for v in range(7)] + [_CHUNK - 16]
    for blk in range(_ROWS_PER_W // _HN_BLK):
        rbase = w * _ROWS_PER_W + blk * _HN_BLK
        pltpu.sync_copy(src2d.at[pl.ds(rbase, _HN_BLK)], src_v)
        pltpu.sync_copy(dst2d.at[pl.ds(rbase, _HN_BLK)], dst_v)
        pltpu.sync_copy(ex2d.at[pl.ds(rbase, _HN_BLK)], ex_v)
        for j in range(_HN_BLK):
            for off in offs:
                s16 = src_v[j, pl.ds(off, 16)]
                e16 = ex_v[j, pl.ds(off, 16)]
                al_v[pl.ds(j * _CHUNK + off, 16)] = \
                    e16 * plsc.load_gather(rec_v, [s16])
        start(0, 0)
        for j in range(_HN_BLK):
            b = j & 1
            if j + 1 < _HN_BLK:
                start(j + 1, (j + 1) & 1)
            descs[b].wait()

            def ebody(i, _):
                sp = plsc.load_gather(al_v, [jnp.full((16,), j * _CHUNK, jnp.int32) + i])
                for t in range(8):
                    rows[b][i, pl.ds(t * 16, 16)] = rows[b][i, pl.ds(t * 16, 16)] * sp
                return 0

            lax.fori_loop(0, _CHUNK, ebody, 0)
            pltpu.sync_copy(rows[b], acc.at[src_v.at[j]], add=True)

    plsc.subcore_barrier()
    pltpu.sync_copy(acc.at[pl.ds(s * rows_per_sub, rows_per_sub)],
                    out.at[c].at[pl.ds(s * rows_per_sub, rows_per_sub)])

    @pl.when(s == 0)
    def _():
        pltpu.sync_copy(acc.at[pl.ds(_NS * rows_per_sub, N - _NS * rows_per_sub)],
                        out.at[c].at[pl.ds(_NS * rows_per_sub, N - _NS * rows_per_sub)])


def _hn_pass(ktab, src2d, dst2d, ex2d, rec, zeros):
    f = pl.kernel(
        _hn_body,
        compiler_params=pltpu.CompilerParams(needs_layout_passes=False),
        out_type=jax.ShapeDtypeStruct((_NC, N, H), jnp.float32),
        mesh=plsc.VectorSubcoreMesh(core_axis_name="c", subcore_axis_name="s"),
        scratch_types=[
            pltpu.VMEM((_HN_BLK, _CHUNK), jnp.int32),
            pltpu.VMEM((_HN_BLK, _CHUNK), jnp.int32),
            pltpu.VMEM((_HN_BLK, _CHUNK), jnp.float32),
            pltpu.VMEM((N,), jnp.float32),
            pltpu.VMEM((_HN_BLK * _CHUNK,), jnp.float32),
            pltpu.VMEM((_CHUNK, H), jnp.float32),
            pltpu.VMEM((_CHUNK, H), jnp.float32),
            pltpu.VMEM_SHARED((N, H), jnp.float32),
            pltpu.SemaphoreType.DMA,
            pltpu.SemaphoreType.DMA,
        ],
    )
    return f(ktab, src2d, dst2d, ex2d, rec, zeros)


_MIN_OWN = 312
_MIN_ACC = 328
_SC_BLK = 4000
_MCAP = 8192


def _min_body(hn, srcf, dstf, out, sb0, db0, sb1, db1, msrc, mdlo,
              rows0, rows1, acc, ssem0, ssem1, gsem0, gsem1):
    c = lax.axis_index("c")
    s = lax.axis_index("s")
    w = s * _NC + c
    lo = w * _MIN_OWN
    hi = jnp.where(w == _NW - 1, M, lo + _MIN_OWN)
    lo16 = jnp.broadcast_to(lo, (16,))
    hi16 = jnp.broadcast_to(hi, (16,))
    sbufs = (sb0, sb1)
    dbufs = (db0, db1)
    ssems = (ssem0, ssem1)
    rbufs = (rows0, rows1)
    gsems = (gsem0, gsem1)

    def zb(v, _):
        msrc[pl.ds(v * 16, 16)] = jnp.zeros((16,), jnp.int32)
        return 0

    lax.fori_loop(0, _MCAP // 16, zb, 0)

    inf16 = jnp.full((16,), jnp.inf, jnp.float32)

    def accinit(v, _):
        acc[v // 8, pl.ds((v % 8) * 16, 16)] = inf16
        return 0

    lax.fori_loop(0, _MIN_ACC * 8, accinit, 0)

    nblk = E // _SC_BLK
    iota16 = lax.iota(jnp.int32, 16)

    def stage_start(blk, b):
        pltpu.async_copy(srcf.at[pl.ds(blk * _SC_BLK, _SC_BLK)], sbufs[b], ssems[b])
        pltpu.async_copy(dstf.at[pl.ds(blk * _SC_BLK, _SC_BLK)], dbufs[b], ssems[b])

    def stage_wait(b):
        pltpu.make_async_copy(srcf.at[pl.ds(0, _SC_BLK)], sbufs[b], ssems[b]).wait()
        pltpu.make_async_copy(dstf.at[pl.ds(0, _SC_BLK)], dbufs[b], ssems[b]).wait()

    def gather_start(cidx, rb):
        pltpu.async_copy(hn.at[msrc.at[pl.ds(cidx * 128, 128)]],
                         rbufs[rb], gsems[rb])

    def gather_wait(rb):
        pltpu.make_async_copy(hn.at[msrc.at[pl.ds(0, 128)]],
                              rbufs[rb], gsems[rb]).wait()

    def rmw_edge(cidx, i, rref):
        dl = plsc.load_gather(mdlo, [jnp.full((16,), 0, jnp.int32)
                                     + (cidx * 128 + i)])
        for t in range(8):
            col = iota16 + (t * 16)
            cur = plsc.load_gather(acc, [dl, col])
            r = rref[i, pl.ds(t * 16, 16)]
            plsc.store_scatter(acc, [dl, col], jnp.minimum(cur, r))

    def rmw_full(cidx, rb):
        def e4(i, _):
            for k in range(4):
                rmw_edge(cidx, 4 * i + k, rbufs[rb])
            return 0

        lax.fori_loop(0, 32, e4, 0)

    def process_full(cnt):
        nch = cnt // 128
        gather_start(0, 0)

        def c2(p2, _):
            c0 = 2 * p2
            c1 = c0 + 1
            gather_start(c1, 1)
            gather_wait(0)
            rmw_full(c0, 0)
            gather_start(c0 + 2, 0)

            gather_wait(1)

            @pl.when(c1 < nch)
            def _():
                rmw_full(c1, 1)

            return 0

        lax.fori_loop(0, (nch + 1) // 2, c2, 0)
        gather_wait(0)
        for k in range(8):
            msrc[pl.ds(k * 16, 16)] = msrc[pl.ds(nch * 128 + k * 16, 16)]
            mdlo[pl.ds(k * 16, 16)] = mdlo[pl.ds(nch * 128 + k * 16, 16)]
        return cnt - nch * 128

    def scan(b, cnt0):
        def scan_body(v, cnt):
            ms, pcs = [], []
            for k in range(5):
                s16 = sbufs[b][pl.ds((5 * v + k) * 16, 16)]
                d16 = dbufs[b][pl.ds((5 * v + k) * 16, 16)]
                m = jnp.logical_and(d16 >= lo16, d16 < hi16)
                ms.append((s16, d16, m))
                pcs.append(jnp.sum(m.astype(jnp.int32)))
            off = cnt
            for k in range(5):
                s16, d16, m = ms[k]
                plsc.store_compressed(msrc.at[pl.ds(off, 16)], s16, mask=m)
                plsc.store_compressed(mdlo.at[pl.ds(off, 16)], d16 - lo16, mask=m)
                off = off + pcs[k]
            return off

        return lax.fori_loop(0, _SC_BLK // 80, scan_body, cnt0)

    stage_start(0, 0)
    _FLUSH = _MCAP - _SC_BLK

    def maybe_flush(cnt):
        return lax.cond(cnt >= _FLUSH, process_full, lambda c: c, cnt)

    def blk2(p, cnt):
        b0 = 2 * p
        stage_start(b0 + 1, 1)
        stage_wait(0)
        cnt = maybe_flush(scan(0, cnt))

        @pl.when(b0 + 2 < nblk)
        def _():
            stage_start(b0 + 2, 0)

        stage_wait(1)
        cnt = maybe_flush(scan(1, cnt))
        return cnt

    cnt = lax.fori_loop(0, nblk // 2, blk2, 0)

    def fchunk(cidx, cnt):
        pltpu.async_copy(hn.at[msrc.at[pl.ds(cidx * 128, 128)]],
                         rows0, gsem0).wait()
        nb = jnp.minimum(cnt - cidx * 128, 128)

        def e1(i, _):
            rmw_edge(cidx, i, rows0)
            return 0

        lax.fori_loop(0, nb, e1, 0)
        return cnt

    lax.fori_loop(0, (cnt + 127) // 128, fchunk, cnt)

    def fin(v, _):
        r = v // 8
        off = (v % 8) * 16
        val = acc[r, pl.ds(off, 16)]
        acc[r, pl.ds(off, 16)] = jnp.where(val * 0.0 == 0.0, val, 0.0)
        return 0

    lax.fori_loop(0, _MIN_ACC * 8, fin, 0)
    pltpu.sync_copy(acc.at[pl.ds(0, _MIN_OWN)], out.at[pl.ds(lo, _MIN_OWN)])

    @pl.when(w == _NW - 1)
    def _():
        pltpu.sync_copy(acc.at[pl.ds(_MIN_OWN, _MIN_ACC - _MIN_OWN)],
                        out.at[pl.ds(M - (_MIN_ACC - _MIN_OWN), _MIN_ACC - _MIN_OWN)])


def _min_pass(hn, srcf, dstf):
    f = pl.kernel(
        _min_body,
        compiler_params=pltpu.CompilerParams(needs_layout_passes=False),
        out_type=jax.ShapeDtypeStruct((M, H), jnp.float32),
        mesh=plsc.VectorSubcoreMesh(core_axis_name="c", subcore_axis_name="s"),
        scratch_types=[
            pltpu.VMEM((_SC_BLK,), jnp.int32),
            pltpu.VMEM((_SC_BLK,), jnp.int32),
            pltpu.VMEM((_SC_BLK,), jnp.int32),
            pltpu.VMEM((_SC_BLK,), jnp.int32),
            pltpu.VMEM((_MCAP,), jnp.int32),
            pltpu.VMEM((_MCAP,), jnp.int32),
            pltpu.VMEM((128, H), jnp.float32),
            pltpu.VMEM((128, H), jnp.float32),
            pltpu.VMEM((_MIN_ACC, H), jnp.float32),
            pltpu.SemaphoreType.DMA,
            pltpu.SemaphoreType.DMA,
            pltpu.SemaphoreType.DMA,
            pltpu.SemaphoreType.DMA,
        ],
    )
    return f(hn, srcf, dstf)


@functools.partial(jax.jit, static_argnums=())
def _seg_sum(table, gidx_rows, sidx_rows, zeros):
    f = pl.kernel(
        _seg_sum_body,
        out_type=jax.ShapeDtypeStruct((_NC, M, H), jnp.float32),
        mesh=plsc.VectorSubcoreMesh(core_axis_name="c", subcore_axis_name="s"),
        scratch_types=[
            pltpu.VMEM((_IDX_BLOCK, _CHUNK), jnp.int32),
            pltpu.VMEM((_IDX_BLOCK, _CHUNK), jnp.int32),
            pltpu.VMEM((_CHUNK, H), jnp.float32),
            pltpu.VMEM((_CHUNK, H), jnp.float32),
            pltpu.VMEM_SHARED((M, H), jnp.float32),
            pltpu.SemaphoreType.DMA,
            pltpu.SemaphoreType.DMA,
        ],
    )
    return f(table, gidx_rows, sidx_rows, zeros)


def kernel(x, x_struct, x_e, edge_index, W1e, b1e, W2e, b2e, W1n, b1n, W2n, b2n, Wq, bq, Wk, bk, att, g1, be1, a1, Wf, bf, g2, be2, a2, Wc1, bc1, Wc2, bc2):
    src = edge_index[0]
    dst = edge_index[1]
    src_rows = src.reshape(E // _CHUNK, _CHUNK)
    dst_rows = dst.reshape(E // _CHUNK, _CHUNK)
    zeros_mh = jnp.zeros((M, H), jnp.float32)

    xe, h1 = _tc_pre(x_e, x, W1e, b1e, W2e, b2e, W1n, b1n)
    degep = _deg(dst).reshape(_NW, M).T
    degnp = _deg(src).reshape(_NW, N).T
    p = _seg_sum(h1, src_rows, dst_rows, zeros_mh)
    e_agg = _tc_agg_div(p, degep)
    p = _seg_sum(e_agg, dst_rows, src_rows, zeros_mh)
    k, sq, sk = _tc_mid(h1, p, degnp, xe, W2n, b2n, Wq, bq, Wk, bk, att)
    sq = sq.reshape(N)
    sk = sk.reshape(M)
    bv = _tc_bv(sq.reshape(8, N // 8), sk.reshape(8, M // 8))
    ex, denp = _attn_ex(sq, sk, src, dst, bv)
    rec = _tc_rec(denp.reshape(_NW, N))
    ex_rows = ex.reshape(E // _CHUNK, _CHUNK)
    p = _hn_pass(k, src_rows, dst_rows, ex_rows, rec, zeros_mh)
    hn = _tc_add2(p)
    hm = _min_pass(hn, src, dst)
    st1 = _tc_stats(hm, xe)
    u, st2 = _tc_fuse(hm, xe, st1, g1, be1, a1, Wf, bf)
    return _tc_out(u, st2, g2, be2, a2, Wc1, bc1, Wc2, bc2)
